# baseline (speedup 1.0000x reference)
import functools
import jax
import jax.numpy as jnp
import numpy as np
from jax import lax
from jax.experimental import pallas as pl
from jax.experimental.pallas import tpu as pltpu
from jax.experimental.pallas import tpu_sc as plsc

H = 128
NTYPE = 4
NC = 2
NS = 16
NW = NC * NS
LANES = 16


def _silu(x):
    return x * jax.nn.sigmoid(x)


def _k1_body(h_ref, wa_ref, wb_ref, a_ref, b_ref):
    for t in range(NTYPE):
        ht = h_ref[:, t, :]
        a_ref[:, t * 2 * H:(t + 1) * 2 * H] = jnp.dot(
            ht, wa_ref[t], preferred_element_type=jnp.float32
        ).astype(jnp.bfloat16)
        b_ref[:, t * 2 * H:(t + 1) * 2 * H] = jnp.dot(
            ht, wb_ref[t], preferred_element_type=jnp.float32
        ).astype(jnp.bfloat16)


def _k1(h, WA, WB, n, bn):
    grid = n // bn
    return pl.pallas_call(
        _k1_body,
        grid=(grid,),
        in_specs=[
            pl.BlockSpec((bn, NTYPE, H), lambda i: (i, 0, 0)),
            pl.BlockSpec((NTYPE, H, 2 * H), lambda i: (0, 0, 0)),
            pl.BlockSpec((NTYPE, H, 2 * H), lambda i: (0, 0, 0)),
        ],
        out_specs=[
            pl.BlockSpec((bn, 8 * H), lambda i: (i, 0)),
            pl.BlockSpec((bn, 8 * H), lambda i: (i, 0)),
        ],
        out_shape=[
            jax.ShapeDtypeStruct((n, 8 * H), jnp.bfloat16),
            jax.ShapeDtypeStruct((n, 8 * H), jnp.bfloat16),
        ],
    )(h, WA, WB)


def _k2_body(a_hbm, b_hbm, src_hbm, dst_hbm, x1_hbm,
             idx_s, idx_d, buf_a, buf_b, sem_a, sem_b, wsem):
    E = src_hbm.shape[0]
    ew = E // NW
    C = 80
    D = 8 * H
    nchunk = ew // C
    wid = lax.axis_index("s") * NC + lax.axis_index("c")
    base = wid * ew

    def chunk(i, carry):
        del carry
        off = base + i * C
        pltpu.sync_copy(src_hbm.at[pl.ds(off, C)], idx_s)
        pltpu.sync_copy(dst_hbm.at[pl.ds(off, C)], idx_d)
        cp_a = pltpu.async_copy(a_hbm.at[idx_s], buf_a, sem_a)
        cp_b = pltpu.async_copy(b_hbm.at[idx_d], buf_b, sem_b)
        cp_a.wait()
        cp_b.wait()

        def row_add(r, carry2):
            del carry2
            for c in range(D // (2 * LANES)):
                sl = pl.ds(c * 2 * LANES, 2 * LANES)
                buf_a[r, sl] = buf_a[r, sl] + buf_b[r, sl]
            return 0

        lax.fori_loop(0, C, row_add, 0)
        pltpu.sync_copy(buf_a, x1_hbm.at[pl.ds(off, C)])
        return 0

    lax.fori_loop(0, nchunk, chunk, 0)


def _k2(A, B, src, dst, e):
    C = 80
    kern = pl.kernel(
        _k2_body,
        out_type=jax.ShapeDtypeStruct((e, 8 * H), jnp.bfloat16),
        mesh=plsc.VectorSubcoreMesh(core_axis_name="c", subcore_axis_name="s"),
        compiler_params=pltpu.CompilerParams(
            needs_layout_passes=False, use_tc_tiling_on_sc=False),
        scratch_types=[
            pltpu.VMEM((C,), jnp.int32),
            pltpu.VMEM((C,), jnp.int32),
            pltpu.VMEM((C, 8 * H), jnp.bfloat16),
            pltpu.VMEM((C, 8 * H), jnp.bfloat16),
            pltpu.SemaphoreType.DMA,
            pltpu.SemaphoreType.DMA,
            pltpu.SemaphoreType.DMA,
        ],
    )
    return kern(A, B, src, dst)


def _k3_body(x1_hbm, ea_ref, el_ref, m_ref, w1ea_ref, b1_ref,
             w2_ref, b2_ref, w3_ref, b3_ref, w4_ref, b4_ref, s_ref,
             xbuf, dsem):
    be = xbuf.shape[0]
    i = pl.program_id(0)
    pltpu.make_async_copy(x1_hbm.at[pl.ds(i * be, be)], xbuf, dsem).start()
    ea = ea_ref[0, :]
    pltpu.make_async_copy(x1_hbm.at[pl.ds(i * be, be)], xbuf, dsem).wait()
    x = (xbuf[...].astype(jnp.float32)
         + ea[:, None] * w1ea_ref[0, :][None, :] + b1_ref[0, :][None, :])
    x = _silu(x)
    inv_el = 1.0 / el_ref[0, :]
    for t in range(NTYPE):
        xt = x[:, t * 2 * H:(t + 1) * 2 * H]
        y = _silu(jnp.dot(xt, w2_ref[t], preferred_element_type=jnp.float32)
                  + b2_ref[t][None, :])
        y = _silu(jnp.dot(y, w3_ref[t], preferred_element_type=jnp.float32)
                  + b3_ref[t][None, :])
        y = jnp.sum(y * w4_ref[t][None, :], axis=-1) + b4_ref[0, t]
        s = jnp.tanh(y) * 10.0
        s_ref[t, :] = s * m_ref[t, :] * inv_el


def _k3(X1, ea2, el2, masks, w1ea2, b12, W2, b2, W3, b3, W4, b4, e):
    be = 512
    grid = e // be
    return pl.pallas_call(
        _k3_body,
        grid=(grid,),
        in_specs=[
            pl.BlockSpec(memory_space=pl.ANY),
            pl.BlockSpec((1, be), lambda i: (0, i)),
            pl.BlockSpec((1, be), lambda i: (0, i)),
            pl.BlockSpec((NTYPE, be), lambda i: (0, i)),
            pl.BlockSpec((1, 8 * H), lambda i: (0, 0)),
            pl.BlockSpec((1, 8 * H), lambda i: (0, 0)),
            pl.BlockSpec((NTYPE, 2 * H, H), lambda i: (0, 0, 0)),
            pl.BlockSpec((NTYPE, H), lambda i: (0, 0)),
            pl.BlockSpec((NTYPE, H, H // 2), lambda i: (0, 0, 0)),
            pl.BlockSpec((NTYPE, H // 2), lambda i: (0, 0)),
            pl.BlockSpec((NTYPE, H // 2), lambda i: (0, 0)),
            pl.BlockSpec((1, NTYPE), lambda i: (0, 0)),
        ],
        out_specs=pl.BlockSpec((NTYPE, be), lambda i: (0, i)),
        out_shape=jax.ShapeDtypeStruct((NTYPE, e), jnp.float32),
        scratch_shapes=[
            pltpu.VMEM((be, 8 * H), jnp.bfloat16),
            pltpu.SemaphoreType.DMA,
        ],
    )(X1, ea2, el2, masks, w1ea2, b12, W2, b2, W3, b3, W4, b4)


def _k4_body(src_hbm, dst_hbm, ea_hbm, m_hbm, s_hbm, coord_hbm,
             eq_hbm, mp_hbm,
             cx, cy, cz, sbuf, dbuf, eabuf, mbuf, scbuf,
             updA, updB, updM, zrow,
             eq_sh, ms_sh, md_sh, sem):
    E = src_hbm.shape[0]
    N = cx.shape[0]
    ew = E // NW
    C = 80
    nchunk = ew // C
    nv = C // LANES
    cid = lax.axis_index("c")
    sid = lax.axis_index("s")
    wid = sid * NC + cid
    base = wid * ew
    RPT = (N // NS) // 8 * 8
    RLAST = N - (NS - 1) * RPT

    z16 = jnp.zeros((LANES,), jnp.float32)

    def zrow_init(i, carry):
        del carry
        zrow[i, :] = z16
        return 0

    lax.fori_loop(0, RLAST, zrow_init, 0)

    def zbuf_init(i, carry):
        del carry
        updA[i, :] = z16
        updB[i, :] = z16
        updM[i, :] = z16
        return 0

    lax.fori_loop(0, C, zbuf_init, 0)

    r0 = sid * RPT

    @pl.when(sid < NS - 1)
    def _():
        for sh in (eq_sh, ms_sh, md_sh):
            pltpu.sync_copy(zrow.at[pl.ds(0, RPT)], sh.at[pl.ds(r0, RPT)])

    @pl.when(sid == NS - 1)
    def _():
        for sh in (eq_sh, ms_sh, md_sh):
            pltpu.sync_copy(zrow, sh.at[pl.ds(r0, RLAST)])

    pltpu.sync_copy(coord_hbm.at[pl.ds(0, N)], cx)
    pltpu.sync_copy(coord_hbm.at[pl.ds(N, N)], cy)
    pltpu.sync_copy(coord_hbm.at[pl.ds(2 * N, N)], cz)
    plsc.subcore_barrier()

    def chunk(i, carry):
        del carry
        off = base + i * C
        pltpu.sync_copy(src_hbm.at[pl.ds(off, C)], sbuf)
        pltpu.sync_copy(dst_hbm.at[pl.ds(off, C)], dbuf)
        pltpu.sync_copy(ea_hbm.at[pl.ds(off, C)], eabuf)
        for t in range(NTYPE):
            pltpu.sync_copy(m_hbm.at[pl.ds(t * E + off, C)],
                            mbuf.at[pl.ds(t * C, C)])
            pltpu.sync_copy(s_hbm.at[pl.ds(t * E + off, C)],
                            scbuf.at[pl.ds(t * C, C)])
        for v in range(nv):
            sl = pl.ds(v * LANES, LANES)
            s16 = sbuf[sl]
            d16 = dbuf[sl]
            ea16 = eabuf[sl]
            ddx = plsc.load_gather(cx, [s16]) - plsc.load_gather(cx, [d16])
            ddy = plsc.load_gather(cy, [s16]) - plsc.load_gather(cy, [d16])
            ddz = plsc.load_gather(cz, [s16]) - plsc.load_gather(cz, [d16])
            row16 = lax.iota(jnp.int32, LANES) + (v * LANES)
            for t in range(NTYPE):
                st = scbuf[pl.ds(t * C + v * LANES, LANES)]
                ct = jnp.full((LANES,), 4 * t, jnp.int32)
                vx = ddx * st
                vy = ddy * st
                vz = ddz * st
                plsc.store_scatter(updA, [row16, ct], vx)
                plsc.store_scatter(updA, [row16, ct + 1], vy)
                plsc.store_scatter(updA, [row16, ct + 2], vz)
                plsc.store_scatter(updB, [row16, ct], -vx)
                plsc.store_scatter(updB, [row16, ct + 1], -vy)
                plsc.store_scatter(updB, [row16, ct + 2], -vz)
                wt = mbuf[pl.ds(t * C + v * LANES, LANES)]
                ctm = jnp.full((LANES,), t, jnp.int32)
                plsc.store_scatter(updM, [row16, ctm], wt)
                plsc.store_scatter(updM, [row16, ctm + NTYPE], wt * ea16)
        pltpu.sync_copy(updA, eq_sh.at[sbuf], add=True)
        pltpu.sync_copy(updB, eq_sh.at[dbuf], add=True)
        pltpu.sync_copy(updM, ms_sh.at[sbuf], add=True)
        pltpu.sync_copy(updM, md_sh.at[dbuf], add=True)
        return 0

    lax.fori_loop(0, nchunk, chunk, 0)
    plsc.subcore_barrier()

    pairs = ((eq_sh, eq_hbm, cid * N), (ms_sh, mp_hbm, 2 * cid * N),
             (md_sh, mp_hbm, (2 * cid + 1) * N))

    @pl.when(sid < NS - 1)
    def _():
        for sh, ob, o0 in pairs:
            pltpu.sync_copy(sh.at[pl.ds(r0, RPT)], ob.at[pl.ds(o0 + r0, RPT)])

    @pl.when(sid == NS - 1)
    def _():
        for sh, ob, o0 in pairs:
            pltpu.sync_copy(sh.at[pl.ds(r0, RLAST)],
                            ob.at[pl.ds(o0 + r0, RLAST)])


def _k4(src, dst, ea, masks, S, coordT, n, e):
    C = 80
    rows_per_tile = n - (NS - 1) * ((n // NS) // 8 * 8)
    kern = pl.kernel(
        _k4_body,
        out_type=[
            jax.ShapeDtypeStruct((NC * n, LANES), jnp.float32),
            jax.ShapeDtypeStruct((2 * NC * n, LANES), jnp.float32),
        ],
        mesh=plsc.VectorSubcoreMesh(core_axis_name="c", subcore_axis_name="s"),
        compiler_params=pltpu.CompilerParams(
            needs_layout_passes=False, use_tc_tiling_on_sc=False),
        scratch_types=[
            pltpu.VMEM((n,), jnp.float32),
            pltpu.VMEM((n,), jnp.float32),
            pltpu.VMEM((n,), jnp.float32),
            pltpu.VMEM((C,), jnp.int32),
            pltpu.VMEM((C,), jnp.int32),
            pltpu.VMEM((C,), jnp.float32),
            pltpu.VMEM((NTYPE * C,), jnp.float32),
            pltpu.VMEM((NTYPE * C,), jnp.float32),
            pltpu.VMEM((C, LANES), jnp.float32),
            pltpu.VMEM((C, LANES), jnp.float32),
            pltpu.VMEM((C, LANES), jnp.float32),
            pltpu.VMEM((rows_per_tile, LANES), jnp.float32),
            pltpu.VMEM_SHARED((n, LANES), jnp.float32),
            pltpu.VMEM_SHARED((n, LANES), jnp.float32),
            pltpu.VMEM_SHARED((n, LANES), jnp.float32),
            pltpu.SemaphoreType.DMA,
        ],
    )
    eq_f, mp_f = kern(src, dst, ea, masks.reshape(-1), S.reshape(-1),
                      coordT.reshape(-1))
    return (eq_f.reshape(NC, n, LANES), mp_f.reshape(2 * NC, n, LANES))


def _ln(x, g_ref, b_ref):
    m = jnp.mean(x, axis=-1, keepdims=True)
    v = jnp.mean(jnp.square(x - m), axis=-1, keepdims=True)
    return (x - m) / jnp.sqrt(v + 1e-5) * g_ref[0, :][None, :] + b_ref[0, :][None, :]


def _k5_body(h_ref, coord_ref, eq_ref, mp_ref, inw_ref, inb_ref,
             lrefs, outw_ref, co_ref, nep_ref):
    hd = H // 4
    mp_src = mp_ref[0] + mp_ref[2]
    mp_dst = mp_ref[1] + mp_ref[3]
    xs = []
    for t in range(NTYPE):
        ht = h_ref[:, t, :]
        e0 = mp_src[:, NTYPE + t] / jnp.maximum(mp_src[:, t], 1.0)
        e1 = mp_dst[:, NTYPE + t] / jnp.maximum(mp_dst[:, t], 1.0)
        x = (jnp.dot(ht, inw_ref[0:H, :], preferred_element_type=jnp.float32)
             + e0[:, None] * inw_ref[H, :][None, :]
             + e1[:, None] * inw_ref[H + 1, :][None, :]
             + inb_ref[0, :][None, :])
        xs.append(x)
    for (wq, bq, wk, bk, wv, bv, wo, bo, f1, f1b, f2, f2b,
         g1, b1, g2, b2) in lrefs:
        qs = [jnp.dot(x, wq[...], preferred_element_type=jnp.float32) + bq[0, :][None, :]
              for x in xs]
        ks = [jnp.dot(x, wk[...], preferred_element_type=jnp.float32) + bk[0, :][None, :]
              for x in xs]
        vs = [jnp.dot(x, wv[...], preferred_element_type=jnp.float32) + bv[0, :][None, :]
              for x in xs]
        os_ = []
        for i in range(NTYPE):
            heads = []
            for hh in range(4):
                c0, c1 = hh * hd, (hh + 1) * hd
                a = [jnp.sum(qs[i][:, c0:c1] * ks[j][:, c0:c1], axis=-1) / np.sqrt(hd)
                     for j in range(NTYPE)]
                mx = jnp.maximum(jnp.maximum(a[0], a[1]), jnp.maximum(a[2], a[3]))
                ex = [jnp.exp(aj - mx) for aj in a]
                den = ex[0] + ex[1] + ex[2] + ex[3]
                oh = sum(ex[j][:, None] * vs[j][:, c0:c1] for j in range(NTYPE))
                heads.append(oh / den[:, None])
            os_.append(jnp.concatenate(heads, axis=1))
        xs = [_ln(xs[i] + jnp.dot(os_[i], wo[...],
                                  preferred_element_type=jnp.float32)
                  + bo[0, :][None, :], g1, b1)
              for i in range(NTYPE)]
        xs = [_ln(x + jnp.dot(_silu(jnp.dot(x, f1[...],
                                            preferred_element_type=jnp.float32)
                                    + f1b[0, :][None, :]),
                              f2[...], preferred_element_type=jnp.float32)
              + f2b[0, :][None, :], g2, b2)
              for x in xs]
    logits = [jnp.sum(x * outw_ref[0, :][None, :], axis=-1) for x in xs]
    mx = jnp.maximum(jnp.maximum(logits[0], logits[1]),
                     jnp.maximum(logits[2], logits[3]))
    ex = [jnp.exp(l - mx) for l in logits]
    den = ex[0] + ex[1] + ex[2] + ex[3]
    wt = [e / den for e in ex]

    eqs = eq_ref[0] + eq_ref[1]
    outc = []
    for c in range(3):
        nep_c = sum(wt[t] * eqs[:, 4 * t + c] for t in range(NTYPE))
        outc.append(nep_c[:, None])
    nep = jnp.concatenate(outc, axis=1)
    co_ref[...] = coord_ref[...] + nep
    nep_ref[...] = nep


def _k5(h, coord, EQ, MP, ap, n):
    bn = 1000
    grid = n // bn
    full2 = lambda shp: pl.BlockSpec(shp, lambda i: (0, 0))

    layer_inputs = []
    layer_specs = []
    for lp in ap['layers']:
        for nm in ('Wq', 'Wk', 'Wv', 'Wo'):
            layer_inputs += [lp[nm], lp[nm + '_b'].reshape(1, H)]
            layer_specs += [full2((H, H)), full2((1, H))]
        layer_inputs += [lp['F1'], lp['F1_b'].reshape(1, 2 * H),
                         lp['F2'], lp['F2_b'].reshape(1, H),
                         lp['ln1_g'].reshape(1, H), lp['ln1_b'].reshape(1, H),
                         lp['ln2_g'].reshape(1, H), lp['ln2_b'].reshape(1, H)]
        layer_specs += [full2((H, 2 * H)), full2((1, 2 * H)),
                        full2((2 * H, H)), full2((1, H)),
                        full2((1, H)), full2((1, H)),
                        full2((1, H)), full2((1, H))]

    nlayer_refs = 16

    def body(*refs):
        h_ref, coord_ref, eq_ref, mp_ref, inw_ref, inb_ref = refs[:6]
        lref_flat = refs[6:6 + len(layer_specs)]
        outw_ref = refs[6 + len(layer_specs)]
        co_ref, nep_ref = refs[-2:]
        lrefs = [tuple(lref_flat[i * nlayer_refs:(i + 1) * nlayer_refs])
                 for i in range(len(ap['layers']))]
        _k5_body(h_ref, coord_ref, eq_ref, mp_ref, inw_ref, inb_ref,
                 lrefs, outw_ref, co_ref, nep_ref)

    return pl.pallas_call(
        body,
        grid=(grid,),
        in_specs=[
            pl.BlockSpec((bn, NTYPE, H), lambda i: (i, 0, 0)),
            pl.BlockSpec((bn, 3), lambda i: (i, 0)),
            pl.BlockSpec((NC, bn, LANES), lambda i: (0, i, 0)),
            pl.BlockSpec((2 * NC, bn, LANES), lambda i: (0, i, 0)),
            pl.BlockSpec((H + 2, H), lambda i: (0, 0)),
            full2((1, H)),
        ] + layer_specs + [full2((1, H))],
        out_specs=[
            pl.BlockSpec((bn, 3), lambda i: (i, 0)),
            pl.BlockSpec((bn, 3), lambda i: (i, 0)),
        ],
        out_shape=[
            jax.ShapeDtypeStruct((n, 3), jnp.float32),
            jax.ShapeDtypeStruct((n, 3), jnp.float32),
        ],
    )(h, coord, EQ, MP, ap['in_W'], ap['in_b'].reshape(1, H),
      *layer_inputs, ap['out_W'].reshape(1, H))


def kernel(h, coord, edge_index, coord_diff, edge_attr, edge_mask,
           edge_length, N, params):
    del coord_diff
    n = h.shape[0]
    e = edge_index.shape[1]
    src = edge_index[0]
    dst = edge_index[1]
    masks = edge_mask.astype(jnp.float32)
    ea = edge_attr[:, 0]
    el = edge_length[:, 0]
    coordT = coord.T

    names = ('bond', 'angle', 'torsion', 'radius')
    mlps = [params[nm] for nm in names]
    WA = jnp.stack([p[0][0][:H] for p in mlps])
    WB = jnp.stack([p[0][0][H:2 * H] for p in mlps])
    w1ea = jnp.concatenate([p[0][0][2 * H] for p in mlps]).reshape(1, 8 * H)
    b1 = jnp.concatenate([p[0][1] for p in mlps]).reshape(1, 8 * H)
    W2 = jnp.stack([p[1][0] for p in mlps])
    b2 = jnp.stack([p[1][1] for p in mlps])
    W3 = jnp.stack([p[2][0] for p in mlps])
    b3 = jnp.stack([p[2][1] for p in mlps])
    W4 = jnp.stack([p[3][0][:, 0] for p in mlps])
    b4 = jnp.stack([p[3][1][0] for p in mlps]).reshape(1, NTYPE)

    A, B = _k1(h, WA, WB, n, 1000)
    X1 = _k2(A, B, src, dst, e)
    S = _k3(X1, ea.reshape(1, e), el.reshape(1, e), masks,
            w1ea, b1, W2, b2, W3, b3, W4, b4, e)
    EQ, MP = _k4(src, dst, ea, masks, S, coordT, n, e)
    coord_new, nep = _k5(h, coord, EQ, MP, params['attn'], n)
    return coord_new, nep


# speedup vs baseline: 1.1070x; 1.1070x over previous
import functools
import jax
import jax.numpy as jnp
import numpy as np
from jax import lax
from jax.experimental import pallas as pl
from jax.experimental.pallas import tpu as pltpu
from jax.experimental.pallas import tpu_sc as plsc

H = 128
NTYPE = 4
NC = 2
NS = 16
NW = NC * NS
LANES = 16


def _silu(x):
    return x * jax.nn.sigmoid(x)


def _k1_body(h_ref, wa_ref, wb_ref, a_ref, b_ref):
    for t in range(NTYPE):
        ht = h_ref[:, t, :]
        a_ref[:, t * 2 * H:(t + 1) * 2 * H] = jnp.dot(
            ht, wa_ref[t], preferred_element_type=jnp.float32
        ).astype(jnp.bfloat16)
        b_ref[:, t * 2 * H:(t + 1) * 2 * H] = jnp.dot(
            ht, wb_ref[t], preferred_element_type=jnp.float32
        ).astype(jnp.bfloat16)


def _k1(h, WA, WB, n, bn):
    grid = n // bn
    return pl.pallas_call(
        _k1_body,
        grid=(grid,),
        in_specs=[
            pl.BlockSpec((bn, NTYPE, H), lambda i: (i, 0, 0)),
            pl.BlockSpec((NTYPE, H, 2 * H), lambda i: (0, 0, 0)),
            pl.BlockSpec((NTYPE, H, 2 * H), lambda i: (0, 0, 0)),
        ],
        out_specs=[
            pl.BlockSpec((bn, 8 * H), lambda i: (i, 0)),
            pl.BlockSpec((bn, 8 * H), lambda i: (i, 0)),
        ],
        out_shape=[
            jax.ShapeDtypeStruct((n, 8 * H), jnp.bfloat16),
            jax.ShapeDtypeStruct((n, 8 * H), jnp.bfloat16),
        ],
    )(h, WA, WB)


def _k2_body(a_hbm, b_hbm, src_hbm, dst_hbm, x1_hbm,
             idx_s, idx_d, buf_a, buf_b, sem_a, sem_b, wsem):
    E = src_hbm.shape[0]
    ew = E // NW
    C = 80
    D = 8 * H
    nchunk = ew // C
    wid = lax.axis_index("s") * NC + lax.axis_index("c")
    base = wid * ew

    def chunk(i, carry):
        del carry
        off = base + i * C
        pltpu.sync_copy(src_hbm.at[pl.ds(off, C)], idx_s)
        pltpu.sync_copy(dst_hbm.at[pl.ds(off, C)], idx_d)
        cp_a = pltpu.async_copy(a_hbm.at[idx_s], buf_a, sem_a)
        cp_b = pltpu.async_copy(b_hbm.at[idx_d], buf_b, sem_b)
        cp_a.wait()
        cp_b.wait()

        def row_add(r, carry2):
            del carry2
            for c in range(D // (2 * LANES)):
                sl = pl.ds(c * 2 * LANES, 2 * LANES)
                buf_a[r, sl] = buf_a[r, sl] + buf_b[r, sl]
            return 0

        lax.fori_loop(0, C, row_add, 0)
        pltpu.sync_copy(buf_a, x1_hbm.at[pl.ds(off, C)])
        return 0

    lax.fori_loop(0, nchunk, chunk, 0)


def _k2(A, B, src, dst, e):
    C = 80
    kern = pl.kernel(
        _k2_body,
        out_type=jax.ShapeDtypeStruct((e, 8 * H), jnp.bfloat16),
        mesh=plsc.VectorSubcoreMesh(core_axis_name="c", subcore_axis_name="s"),
        compiler_params=pltpu.CompilerParams(
            needs_layout_passes=False, use_tc_tiling_on_sc=False),
        scratch_types=[
            pltpu.VMEM((C,), jnp.int32),
            pltpu.VMEM((C,), jnp.int32),
            pltpu.VMEM((C, 8 * H), jnp.bfloat16),
            pltpu.VMEM((C, 8 * H), jnp.bfloat16),
            pltpu.SemaphoreType.DMA,
            pltpu.SemaphoreType.DMA,
            pltpu.SemaphoreType.DMA,
        ],
    )
    return kern(A, B, src, dst)


def _k3_body(x1_hbm, ea_ref, el_ref, m_ref, w1ea_ref, b1_ref,
             w2_ref, b2_ref, w3_ref, b3_ref, w4_ref, b4_ref, s_ref,
             xbuf, dsem):
    be = xbuf.shape[1]
    i = pl.program_id(0)
    g = pl.num_programs(0)
    slot = lax.rem(i, 2)
    nslot = lax.rem(i + 1, 2)

    @pl.when(i == 0)
    def _():
        pltpu.make_async_copy(
            x1_hbm.at[pl.ds(0, be)], xbuf.at[0], dsem.at[0]).start()

    @pl.when(i + 1 < g)
    def _():
        pltpu.make_async_copy(
            x1_hbm.at[pl.ds((i + 1) * be, be)], xbuf.at[nslot],
            dsem.at[nslot]).start()

    ea = ea_ref[0, :]
    pltpu.make_async_copy(
        x1_hbm.at[pl.ds(i * be, be)], xbuf.at[slot], dsem.at[slot]).wait()
    x = (xbuf[slot].astype(jnp.float32)
         + ea[:, None] * w1ea_ref[0, :][None, :] + b1_ref[0, :][None, :])
    x = _silu(x)
    inv_el = 1.0 / el_ref[0, :]
    for t in range(NTYPE):
        xt = x[:, t * 2 * H:(t + 1) * 2 * H]
        y = _silu(jnp.dot(xt, w2_ref[t], preferred_element_type=jnp.float32)
                  + b2_ref[t][None, :])
        y = _silu(jnp.dot(y, w3_ref[t], preferred_element_type=jnp.float32)
                  + b3_ref[t][None, :])
        y = jnp.sum(y * w4_ref[t][None, :], axis=-1) + b4_ref[0, t]
        s = jnp.tanh(y) * 10.0
        s_ref[t, :] = s * m_ref[t, :] * inv_el


def _k3(X1, ea2, el2, masks, w1ea2, b12, W2, b2, W3, b3, W4, b4, e):
    be = 512
    grid = e // be
    return pl.pallas_call(
        _k3_body,
        grid=(grid,),
        in_specs=[
            pl.BlockSpec(memory_space=pl.ANY),
            pl.BlockSpec((1, be), lambda i: (0, i)),
            pl.BlockSpec((1, be), lambda i: (0, i)),
            pl.BlockSpec((NTYPE, be), lambda i: (0, i)),
            pl.BlockSpec((1, 8 * H), lambda i: (0, 0)),
            pl.BlockSpec((1, 8 * H), lambda i: (0, 0)),
            pl.BlockSpec((NTYPE, 2 * H, H), lambda i: (0, 0, 0)),
            pl.BlockSpec((NTYPE, H), lambda i: (0, 0)),
            pl.BlockSpec((NTYPE, H, H // 2), lambda i: (0, 0, 0)),
            pl.BlockSpec((NTYPE, H // 2), lambda i: (0, 0)),
            pl.BlockSpec((NTYPE, H // 2), lambda i: (0, 0)),
            pl.BlockSpec((1, NTYPE), lambda i: (0, 0)),
        ],
        out_specs=pl.BlockSpec((NTYPE, be), lambda i: (0, i)),
        out_shape=jax.ShapeDtypeStruct((NTYPE, e), jnp.float32),
        scratch_shapes=[
            pltpu.VMEM((2, be, 8 * H), jnp.bfloat16),
            pltpu.SemaphoreType.DMA((2,)),
        ],
    )(X1, ea2, el2, masks, w1ea2, b12, W2, b2, W3, b3, W4, b4)


def _k4_body(src_hbm, dst_hbm, ea_hbm, m_hbm, s_hbm, coord_hbm,
             eq_hbm, mp_hbm,
             cx, cy, cz, sbuf, dbuf, eabuf, mbuf, scbuf,
             updA, updB, updM, zrow,
             eq_sh, ms_sh, md_sh, sem):
    E = src_hbm.shape[0]
    N = cx.shape[0]
    ew = E // NW
    C = 80
    nchunk = ew // C
    nv = C // LANES
    cid = lax.axis_index("c")
    sid = lax.axis_index("s")
    wid = sid * NC + cid
    base = wid * ew
    RPT = (N // NS) // 8 * 8
    RLAST = N - (NS - 1) * RPT

    z16 = jnp.zeros((LANES,), jnp.float32)

    def zrow_init(i, carry):
        del carry
        zrow[i, :] = z16
        return 0

    lax.fori_loop(0, RLAST, zrow_init, 0)

    def zbuf_init(i, carry):
        del carry
        updA[i, :] = z16
        updB[i, :] = z16
        updM[i, :] = z16
        return 0

    lax.fori_loop(0, C, zbuf_init, 0)

    r0 = sid * RPT

    @pl.when(sid < NS - 1)
    def _():
        for sh in (eq_sh, ms_sh, md_sh):
            pltpu.sync_copy(zrow.at[pl.ds(0, RPT)], sh.at[pl.ds(r0, RPT)])

    @pl.when(sid == NS - 1)
    def _():
        for sh in (eq_sh, ms_sh, md_sh):
            pltpu.sync_copy(zrow, sh.at[pl.ds(r0, RLAST)])

    pltpu.sync_copy(coord_hbm.at[pl.ds(0, N)], cx)
    pltpu.sync_copy(coord_hbm.at[pl.ds(N, N)], cy)
    pltpu.sync_copy(coord_hbm.at[pl.ds(2 * N, N)], cz)
    plsc.subcore_barrier()

    def chunk(i, carry):
        del carry
        off = base + i * C
        pltpu.sync_copy(src_hbm.at[pl.ds(off, C)], sbuf)
        pltpu.sync_copy(dst_hbm.at[pl.ds(off, C)], dbuf)
        pltpu.sync_copy(ea_hbm.at[pl.ds(off, C)], eabuf)
        for t in range(NTYPE):
            pltpu.sync_copy(m_hbm.at[pl.ds(t * E + off, C)],
                            mbuf.at[pl.ds(t * C, C)])
            pltpu.sync_copy(s_hbm.at[pl.ds(t * E + off, C)],
                            scbuf.at[pl.ds(t * C, C)])
        for v in range(nv):
            sl = pl.ds(v * LANES, LANES)
            s16 = sbuf[sl]
            d16 = dbuf[sl]
            ea16 = eabuf[sl]
            ddx = plsc.load_gather(cx, [s16]) - plsc.load_gather(cx, [d16])
            ddy = plsc.load_gather(cy, [s16]) - plsc.load_gather(cy, [d16])
            ddz = plsc.load_gather(cz, [s16]) - plsc.load_gather(cz, [d16])
            row16 = lax.iota(jnp.int32, LANES) + (v * LANES)
            for t in range(NTYPE):
                st = scbuf[pl.ds(t * C + v * LANES, LANES)]
                ct = jnp.full((LANES,), 4 * t, jnp.int32)
                vx = ddx * st
                vy = ddy * st
                vz = ddz * st
                plsc.store_scatter(updA, [row16, ct], vx)
                plsc.store_scatter(updA, [row16, ct + 1], vy)
                plsc.store_scatter(updA, [row16, ct + 2], vz)
                plsc.store_scatter(updB, [row16, ct], -vx)
                plsc.store_scatter(updB, [row16, ct + 1], -vy)
                plsc.store_scatter(updB, [row16, ct + 2], -vz)
                wt = mbuf[pl.ds(t * C + v * LANES, LANES)]
                ctm = jnp.full((LANES,), t, jnp.int32)
                plsc.store_scatter(updM, [row16, ctm], wt)
                plsc.store_scatter(updM, [row16, ctm + NTYPE], wt * ea16)
        pltpu.sync_copy(updA, eq_sh.at[sbuf], add=True)
        pltpu.sync_copy(updB, eq_sh.at[dbuf], add=True)
        pltpu.sync_copy(updM, ms_sh.at[sbuf], add=True)
        pltpu.sync_copy(updM, md_sh.at[dbuf], add=True)
        return 0

    lax.fori_loop(0, nchunk, chunk, 0)
    plsc.subcore_barrier()

    pairs = ((eq_sh, eq_hbm, cid * N), (ms_sh, mp_hbm, 2 * cid * N),
             (md_sh, mp_hbm, (2 * cid + 1) * N))

    @pl.when(sid < NS - 1)
    def _():
        for sh, ob, o0 in pairs:
            pltpu.sync_copy(sh.at[pl.ds(r0, RPT)], ob.at[pl.ds(o0 + r0, RPT)])

    @pl.when(sid == NS - 1)
    def _():
        for sh, ob, o0 in pairs:
            pltpu.sync_copy(sh.at[pl.ds(r0, RLAST)],
                            ob.at[pl.ds(o0 + r0, RLAST)])


def _k4(src, dst, ea, masks, S, coordT, n, e):
    C = 80
    rows_per_tile = n - (NS - 1) * ((n // NS) // 8 * 8)
    kern = pl.kernel(
        _k4_body,
        out_type=[
            jax.ShapeDtypeStruct((NC * n, LANES), jnp.float32),
            jax.ShapeDtypeStruct((2 * NC * n, LANES), jnp.float32),
        ],
        mesh=plsc.VectorSubcoreMesh(core_axis_name="c", subcore_axis_name="s"),
        compiler_params=pltpu.CompilerParams(
            needs_layout_passes=False, use_tc_tiling_on_sc=False),
        scratch_types=[
            pltpu.VMEM((n,), jnp.float32),
            pltpu.VMEM((n,), jnp.float32),
            pltpu.VMEM((n,), jnp.float32),
            pltpu.VMEM((C,), jnp.int32),
            pltpu.VMEM((C,), jnp.int32),
            pltpu.VMEM((C,), jnp.float32),
            pltpu.VMEM((NTYPE * C,), jnp.float32),
            pltpu.VMEM((NTYPE * C,), jnp.float32),
            pltpu.VMEM((C, LANES), jnp.float32),
            pltpu.VMEM((C, LANES), jnp.float32),
            pltpu.VMEM((C, LANES), jnp.float32),
            pltpu.VMEM((rows_per_tile, LANES), jnp.float32),
            pltpu.VMEM_SHARED((n, LANES), jnp.float32),
            pltpu.VMEM_SHARED((n, LANES), jnp.float32),
            pltpu.VMEM_SHARED((n, LANES), jnp.float32),
            pltpu.SemaphoreType.DMA,
        ],
    )
    eq_f, mp_f = kern(src, dst, ea, masks.reshape(-1), S.reshape(-1),
                      coordT.reshape(-1))
    return (eq_f.reshape(NC, n, LANES), mp_f.reshape(2 * NC, n, LANES))


def _ln(x, g_ref, b_ref):
    m = jnp.mean(x, axis=-1, keepdims=True)
    v = jnp.mean(jnp.square(x - m), axis=-1, keepdims=True)
    return (x - m) / jnp.sqrt(v + 1e-5) * g_ref[0, :][None, :] + b_ref[0, :][None, :]


def _k5_body(h_ref, coord_ref, eq_ref, mp_ref, inw_ref, inb_ref,
             lrefs, outw_ref, co_ref, nep_ref):
    hd = H // 4
    mp_src = mp_ref[0] + mp_ref[2]
    mp_dst = mp_ref[1] + mp_ref[3]
    xs = []
    for t in range(NTYPE):
        ht = h_ref[:, t, :]
        e0 = mp_src[:, NTYPE + t] / jnp.maximum(mp_src[:, t], 1.0)
        e1 = mp_dst[:, NTYPE + t] / jnp.maximum(mp_dst[:, t], 1.0)
        x = (jnp.dot(ht, inw_ref[0:H, :], preferred_element_type=jnp.float32)
             + e0[:, None] * inw_ref[H, :][None, :]
             + e1[:, None] * inw_ref[H + 1, :][None, :]
             + inb_ref[0, :][None, :])
        xs.append(x)
    for (wq, bq, wk, bk, wv, bv, wo, bo, f1, f1b, f2, f2b,
         g1, b1, g2, b2) in lrefs:
        qs = [jnp.dot(x, wq[...], preferred_element_type=jnp.float32) + bq[0, :][None, :]
              for x in xs]
        ks = [jnp.dot(x, wk[...], preferred_element_type=jnp.float32) + bk[0, :][None, :]
              for x in xs]
        vs = [jnp.dot(x, wv[...], preferred_element_type=jnp.float32) + bv[0, :][None, :]
              for x in xs]
        os_ = []
        for i in range(NTYPE):
            heads = []
            for hh in range(4):
                c0, c1 = hh * hd, (hh + 1) * hd
                a = [jnp.sum(qs[i][:, c0:c1] * ks[j][:, c0:c1], axis=-1) / np.sqrt(hd)
                     for j in range(NTYPE)]
                mx = jnp.maximum(jnp.maximum(a[0], a[1]), jnp.maximum(a[2], a[3]))
                ex = [jnp.exp(aj - mx) for aj in a]
                den = ex[0] + ex[1] + ex[2] + ex[3]
                oh = sum(ex[j][:, None] * vs[j][:, c0:c1] for j in range(NTYPE))
                heads.append(oh / den[:, None])
            os_.append(jnp.concatenate(heads, axis=1))
        xs = [_ln(xs[i] + jnp.dot(os_[i], wo[...],
                                  preferred_element_type=jnp.float32)
                  + bo[0, :][None, :], g1, b1)
              for i in range(NTYPE)]
        xs = [_ln(x + jnp.dot(_silu(jnp.dot(x, f1[...],
                                            preferred_element_type=jnp.float32)
                                    + f1b[0, :][None, :]),
                              f2[...], preferred_element_type=jnp.float32)
              + f2b[0, :][None, :], g2, b2)
              for x in xs]
    logits = [jnp.sum(x * outw_ref[0, :][None, :], axis=-1) for x in xs]
    mx = jnp.maximum(jnp.maximum(logits[0], logits[1]),
                     jnp.maximum(logits[2], logits[3]))
    ex = [jnp.exp(l - mx) for l in logits]
    den = ex[0] + ex[1] + ex[2] + ex[3]
    wt = [e / den for e in ex]

    eqs = eq_ref[0] + eq_ref[1]
    outc = []
    for c in range(3):
        nep_c = sum(wt[t] * eqs[:, 4 * t + c] for t in range(NTYPE))
        outc.append(nep_c[:, None])
    nep = jnp.concatenate(outc, axis=1)
    co_ref[...] = coord_ref[...] + nep
    nep_ref[...] = nep


def _k5(h, coord, EQ, MP, ap, n):
    bn = 1000
    grid = n // bn
    full2 = lambda shp: pl.BlockSpec(shp, lambda i: (0, 0))

    layer_inputs = []
    layer_specs = []
    for lp in ap['layers']:
        for nm in ('Wq', 'Wk', 'Wv', 'Wo'):
            layer_inputs += [lp[nm], lp[nm + '_b'].reshape(1, H)]
            layer_specs += [full2((H, H)), full2((1, H))]
        layer_inputs += [lp['F1'], lp['F1_b'].reshape(1, 2 * H),
                         lp['F2'], lp['F2_b'].reshape(1, H),
                         lp['ln1_g'].reshape(1, H), lp['ln1_b'].reshape(1, H),
                         lp['ln2_g'].reshape(1, H), lp['ln2_b'].reshape(1, H)]
        layer_specs += [full2((H, 2 * H)), full2((1, 2 * H)),
                        full2((2 * H, H)), full2((1, H)),
                        full2((1, H)), full2((1, H)),
                        full2((1, H)), full2((1, H))]

    nlayer_refs = 16

    def body(*refs):
        h_ref, coord_ref, eq_ref, mp_ref, inw_ref, inb_ref = refs[:6]
        lref_flat = refs[6:6 + len(layer_specs)]
        outw_ref = refs[6 + len(layer_specs)]
        co_ref, nep_ref = refs[-2:]
        lrefs = [tuple(lref_flat[i * nlayer_refs:(i + 1) * nlayer_refs])
                 for i in range(len(ap['layers']))]
        _k5_body(h_ref, coord_ref, eq_ref, mp_ref, inw_ref, inb_ref,
                 lrefs, outw_ref, co_ref, nep_ref)

    return pl.pallas_call(
        body,
        grid=(grid,),
        in_specs=[
            pl.BlockSpec((bn, NTYPE, H), lambda i: (i, 0, 0)),
            pl.BlockSpec((bn, 3), lambda i: (i, 0)),
            pl.BlockSpec((NC, bn, LANES), lambda i: (0, i, 0)),
            pl.BlockSpec((2 * NC, bn, LANES), lambda i: (0, i, 0)),
            pl.BlockSpec((H + 2, H), lambda i: (0, 0)),
            full2((1, H)),
        ] + layer_specs + [full2((1, H))],
        out_specs=[
            pl.BlockSpec((bn, 3), lambda i: (i, 0)),
            pl.BlockSpec((bn, 3), lambda i: (i, 0)),
        ],
        out_shape=[
            jax.ShapeDtypeStruct((n, 3), jnp.float32),
            jax.ShapeDtypeStruct((n, 3), jnp.float32),
        ],
    )(h, coord, EQ, MP, ap['in_W'], ap['in_b'].reshape(1, H),
      *layer_inputs, ap['out_W'].reshape(1, H))


def kernel(h, coord, edge_index, coord_diff, edge_attr, edge_mask,
           edge_length, N, params):
    del coord_diff
    n = h.shape[0]
    e = edge_index.shape[1]
    src = edge_index[0]
    dst = edge_index[1]
    masks = edge_mask.astype(jnp.float32)
    ea = edge_attr[:, 0]
    el = edge_length[:, 0]
    coordT = coord.T

    names = ('bond', 'angle', 'torsion', 'radius')
    mlps = [params[nm] for nm in names]
    WA = jnp.stack([p[0][0][:H] for p in mlps])
    WB = jnp.stack([p[0][0][H:2 * H] for p in mlps])
    w1ea = jnp.concatenate([p[0][0][2 * H] for p in mlps]).reshape(1, 8 * H)
    b1 = jnp.concatenate([p[0][1] for p in mlps]).reshape(1, 8 * H)
    W2 = jnp.stack([p[1][0] for p in mlps])
    b2 = jnp.stack([p[1][1] for p in mlps])
    W3 = jnp.stack([p[2][0] for p in mlps])
    b3 = jnp.stack([p[2][1] for p in mlps])
    W4 = jnp.stack([p[3][0][:, 0] for p in mlps])
    b4 = jnp.stack([p[3][1][0] for p in mlps]).reshape(1, NTYPE)

    A, B = _k1(h, WA, WB, n, 1000)
    X1 = _k2(A, B, src, dst, e)
    S = _k3(X1, ea.reshape(1, e), el.reshape(1, e), masks,
            w1ea, b1, W2, b2, W3, b3, W4, b4, e)
    EQ, MP = _k4(src, dst, ea, masks, S, coordT, n, e)
    coord_new, nep = _k5(h, coord, EQ, MP, params['attn'], n)
    return coord_new, nep


# speedup vs baseline: 1.1936x; 1.0782x over previous
import functools
import jax
import jax.numpy as jnp
import numpy as np
from jax import lax
from jax.experimental import pallas as pl
from jax.experimental.pallas import tpu as pltpu
from jax.experimental.pallas import tpu_sc as plsc

H = 128
NTYPE = 4
NC = 2
NS = 16
NW = NC * NS
LANES = 16


def _silu(x):
    return x * jax.nn.sigmoid(x)


def _k1_body(h_ref, wa_ref, wb_ref, a_ref, b_ref):
    for t in range(NTYPE):
        ht = h_ref[:, t, :]
        a_ref[:, t * 2 * H:(t + 1) * 2 * H] = jnp.dot(
            ht, wa_ref[t], preferred_element_type=jnp.float32)
        b_ref[:, t * 2 * H:(t + 1) * 2 * H] = jnp.dot(
            ht, wb_ref[t], preferred_element_type=jnp.float32)


def _k1(h, WA, WB, n, bn):
    grid = n // bn
    return pl.pallas_call(
        _k1_body,
        grid=(grid,),
        in_specs=[
            pl.BlockSpec((bn, NTYPE, H), lambda i: (i, 0, 0)),
            pl.BlockSpec((NTYPE, H, 2 * H), lambda i: (0, 0, 0)),
            pl.BlockSpec((NTYPE, H, 2 * H), lambda i: (0, 0, 0)),
        ],
        out_specs=[
            pl.BlockSpec((bn, 8 * H), lambda i: (i, 0)),
            pl.BlockSpec((bn, 8 * H), lambda i: (i, 0)),
        ],
        out_shape=[
            jax.ShapeDtypeStruct((n, 8 * H), jnp.float32),
            jax.ShapeDtypeStruct((n, 8 * H), jnp.float32),
        ],
    )(h, WA, WB)


def _k2_body(a_hbm, b_hbm, src_hbm, dst_hbm, x1_hbm,
             idx_s, idx_d, buf_a, buf_b, sem_a, sem_b, wsem):
    E = src_hbm.shape[0]
    ew = E // NW
    C = 40
    D = 8 * H
    nchunk = ew // C
    wid = lax.axis_index("s") * NC + lax.axis_index("c")
    base = wid * ew

    def chunk(i, carry):
        del carry
        off = base + i * C
        pltpu.sync_copy(src_hbm.at[pl.ds(off, C)], idx_s)
        pltpu.sync_copy(dst_hbm.at[pl.ds(off, C)], idx_d)
        cp_a = pltpu.async_copy(a_hbm.at[idx_s], buf_a, sem_a)
        cp_b = pltpu.async_copy(b_hbm.at[idx_d], buf_b, sem_b)
        cp_a.wait()
        cp_b.wait()

        def row_add(r, carry2):
            del carry2
            for c in range(D // LANES):
                sl = pl.ds(c * LANES, LANES)
                buf_a[r, sl] = buf_a[r, sl] + buf_b[r, sl]
            return 0

        lax.fori_loop(0, C, row_add, 0)
        pltpu.sync_copy(buf_a, x1_hbm.at[pl.ds(off, C)])
        return 0

    lax.fori_loop(0, nchunk, chunk, 0)


def _k2(A, B, src, dst, e):
    C = 40
    kern = pl.kernel(
        _k2_body,
        out_type=jax.ShapeDtypeStruct((e, 8 * H), jnp.float32),
        mesh=plsc.VectorSubcoreMesh(core_axis_name="c", subcore_axis_name="s"),
        compiler_params=pltpu.CompilerParams(
            needs_layout_passes=False, use_tc_tiling_on_sc=False),
        scratch_types=[
            pltpu.VMEM((C,), jnp.int32),
            pltpu.VMEM((C,), jnp.int32),
            pltpu.VMEM((C, 8 * H), jnp.float32),
            pltpu.VMEM((C, 8 * H), jnp.float32),
            pltpu.SemaphoreType.DMA,
            pltpu.SemaphoreType.DMA,
            pltpu.SemaphoreType.DMA,
        ],
    )
    return kern(A, B, src, dst)


def _k3_body(x1_ref, ea_ref, el_ref, m_ref, w1ea_ref, b1_ref,
             w2_ref, b2_ref, w3_ref, b3_ref, w4_ref, b4_ref, s_ref):
    ea = ea_ref[0, :]
    x = (x1_ref[...]
         + ea[:, None] * w1ea_ref[0, :][None, :] + b1_ref[0, :][None, :])
    x = _silu(x)
    inv_el = 1.0 / el_ref[0, :]
    for t in range(NTYPE):
        xt = x[:, t * 2 * H:(t + 1) * 2 * H].astype(jnp.bfloat16)
        y = _silu(jnp.dot(xt, w2_ref[t], preferred_element_type=jnp.float32)
                  + b2_ref[t][None, :])
        y = _silu(jnp.dot(y.astype(jnp.bfloat16), w3_ref[t],
                          preferred_element_type=jnp.float32)
                  + b3_ref[t][None, :])
        y = jnp.sum(y * w4_ref[t][None, :], axis=-1) + b4_ref[0, t]
        s = jnp.tanh(y) * 10.0
        s_ref[t, :] = s * m_ref[t, :] * inv_el


def _k3(X1, ea2, el2, masks, w1ea2, b12, W2, b2, W3, b3, W4, b4, e):
    be = 512
    grid = e // be
    return pl.pallas_call(
        _k3_body,
        grid=(grid,),
        in_specs=[
            pl.BlockSpec((be, 8 * H), lambda i: (i, 0)),
            pl.BlockSpec((1, be), lambda i: (0, i)),
            pl.BlockSpec((1, be), lambda i: (0, i)),
            pl.BlockSpec((NTYPE, be), lambda i: (0, i)),
            pl.BlockSpec((1, 8 * H), lambda i: (0, 0)),
            pl.BlockSpec((1, 8 * H), lambda i: (0, 0)),
            pl.BlockSpec((NTYPE, 2 * H, H), lambda i: (0, 0, 0)),
            pl.BlockSpec((NTYPE, H), lambda i: (0, 0)),
            pl.BlockSpec((NTYPE, H, H // 2), lambda i: (0, 0, 0)),
            pl.BlockSpec((NTYPE, H // 2), lambda i: (0, 0)),
            pl.BlockSpec((NTYPE, H // 2), lambda i: (0, 0)),
            pl.BlockSpec((1, NTYPE), lambda i: (0, 0)),
        ],
        out_specs=pl.BlockSpec((NTYPE, be), lambda i: (0, i)),
        out_shape=jax.ShapeDtypeStruct((NTYPE, e), jnp.float32),
    )(X1, ea2, el2, masks, w1ea2, b12, W2, b2, W3, b3, W4, b4)


def _k4_body(src_hbm, dst_hbm, ea_hbm, m_hbm, s_hbm, coord_hbm,
             eq_hbm, mp_hbm,
             cx, cy, cz, sbuf, dbuf, eabuf, mbuf, scbuf,
             updA, updB, updM, zrow,
             eq_sh, ms_sh, md_sh, sem):
    E = src_hbm.shape[0]
    N = cx.shape[0]
    ew = E // NW
    C = 80
    nchunk = ew // C
    nv = C // LANES
    cid = lax.axis_index("c")
    sid = lax.axis_index("s")
    wid = sid * NC + cid
    base = wid * ew
    RPT = (N // NS) // 8 * 8
    RLAST = N - (NS - 1) * RPT

    z16 = jnp.zeros((LANES,), jnp.float32)

    def zrow_init(i, carry):
        del carry
        zrow[i, :] = z16
        return 0

    lax.fori_loop(0, RLAST, zrow_init, 0)

    def zbuf_init(i, carry):
        del carry
        updA[i, :] = z16
        updB[i, :] = z16
        updM[i, :] = z16
        return 0

    lax.fori_loop(0, C, zbuf_init, 0)

    r0 = sid * RPT

    @pl.when(sid < NS - 1)
    def _():
        for sh in (eq_sh, ms_sh, md_sh):
            pltpu.sync_copy(zrow.at[pl.ds(0, RPT)], sh.at[pl.ds(r0, RPT)])

    @pl.when(sid == NS - 1)
    def _():
        for sh in (eq_sh, ms_sh, md_sh):
            pltpu.sync_copy(zrow, sh.at[pl.ds(r0, RLAST)])

    pltpu.sync_copy(coord_hbm.at[pl.ds(0, N)], cx)
    pltpu.sync_copy(coord_hbm.at[pl.ds(N, N)], cy)
    pltpu.sync_copy(coord_hbm.at[pl.ds(2 * N, N)], cz)
    plsc.subcore_barrier()

    def chunk(i, carry):
        del carry
        off = base + i * C
        pltpu.sync_copy(src_hbm.at[pl.ds(off, C)], sbuf)
        pltpu.sync_copy(dst_hbm.at[pl.ds(off, C)], dbuf)
        pltpu.sync_copy(ea_hbm.at[pl.ds(off, C)], eabuf)
        for t in range(NTYPE):
            pltpu.sync_copy(m_hbm.at[pl.ds(t * E + off, C)],
                            mbuf.at[pl.ds(t * C, C)])
            pltpu.sync_copy(s_hbm.at[pl.ds(t * E + off, C)],
                            scbuf.at[pl.ds(t * C, C)])
        for v in range(nv):
            sl = pl.ds(v * LANES, LANES)
            s16 = sbuf[sl]
            d16 = dbuf[sl]
            ea16 = eabuf[sl]
            ddx = plsc.load_gather(cx, [s16]) - plsc.load_gather(cx, [d16])
            ddy = plsc.load_gather(cy, [s16]) - plsc.load_gather(cy, [d16])
            ddz = plsc.load_gather(cz, [s16]) - plsc.load_gather(cz, [d16])
            row16 = lax.iota(jnp.int32, LANES) + (v * LANES)
            for t in range(NTYPE):
                st = scbuf[pl.ds(t * C + v * LANES, LANES)]
                ct = jnp.full((LANES,), 4 * t, jnp.int32)
                vx = ddx * st
                vy = ddy * st
                vz = ddz * st
                plsc.store_scatter(updA, [row16, ct], vx)
                plsc.store_scatter(updA, [row16, ct + 1], vy)
                plsc.store_scatter(updA, [row16, ct + 2], vz)
                plsc.store_scatter(updB, [row16, ct], -vx)
                plsc.store_scatter(updB, [row16, ct + 1], -vy)
                plsc.store_scatter(updB, [row16, ct + 2], -vz)
                wt = mbuf[pl.ds(t * C + v * LANES, LANES)]
                ctm = jnp.full((LANES,), t, jnp.int32)
                plsc.store_scatter(updM, [row16, ctm], wt)
                plsc.store_scatter(updM, [row16, ctm + NTYPE], wt * ea16)
        pltpu.sync_copy(updA, eq_sh.at[sbuf], add=True)
        pltpu.sync_copy(updB, eq_sh.at[dbuf], add=True)
        pltpu.sync_copy(updM, ms_sh.at[sbuf], add=True)
        pltpu.sync_copy(updM, md_sh.at[dbuf], add=True)
        return 0

    lax.fori_loop(0, nchunk, chunk, 0)
    plsc.subcore_barrier()

    pairs = ((eq_sh, eq_hbm, cid * N), (ms_sh, mp_hbm, 2 * cid * N),
             (md_sh, mp_hbm, (2 * cid + 1) * N))

    @pl.when(sid < NS - 1)
    def _():
        for sh, ob, o0 in pairs:
            pltpu.sync_copy(sh.at[pl.ds(r0, RPT)], ob.at[pl.ds(o0 + r0, RPT)])

    @pl.when(sid == NS - 1)
    def _():
        for sh, ob, o0 in pairs:
            pltpu.sync_copy(sh.at[pl.ds(r0, RLAST)],
                            ob.at[pl.ds(o0 + r0, RLAST)])


def _k4(src, dst, ea, masks, S, coordT, n, e):
    C = 80
    rows_per_tile = n - (NS - 1) * ((n // NS) // 8 * 8)
    kern = pl.kernel(
        _k4_body,
        out_type=[
            jax.ShapeDtypeStruct((NC * n, LANES), jnp.float32),
            jax.ShapeDtypeStruct((2 * NC * n, LANES), jnp.float32),
        ],
        mesh=plsc.VectorSubcoreMesh(core_axis_name="c", subcore_axis_name="s"),
        compiler_params=pltpu.CompilerParams(
            needs_layout_passes=False, use_tc_tiling_on_sc=False),
        scratch_types=[
            pltpu.VMEM((n,), jnp.float32),
            pltpu.VMEM((n,), jnp.float32),
            pltpu.VMEM((n,), jnp.float32),
            pltpu.VMEM((C,), jnp.int32),
            pltpu.VMEM((C,), jnp.int32),
            pltpu.VMEM((C,), jnp.float32),
            pltpu.VMEM((NTYPE * C,), jnp.float32),
            pltpu.VMEM((NTYPE * C,), jnp.float32),
            pltpu.VMEM((C, LANES), jnp.float32),
            pltpu.VMEM((C, LANES), jnp.float32),
            pltpu.VMEM((C, LANES), jnp.float32),
            pltpu.VMEM((rows_per_tile, LANES), jnp.float32),
            pltpu.VMEM_SHARED((n, LANES), jnp.float32),
            pltpu.VMEM_SHARED((n, LANES), jnp.float32),
            pltpu.VMEM_SHARED((n, LANES), jnp.float32),
            pltpu.SemaphoreType.DMA,
        ],
    )
    eq_f, mp_f = kern(src, dst, ea, masks.reshape(-1), S.reshape(-1),
                      coordT.reshape(-1))
    return (eq_f.reshape(NC, n, LANES), mp_f.reshape(2 * NC, n, LANES))


def _ln(x, g_ref, b_ref):
    m = jnp.mean(x, axis=-1, keepdims=True)
    v = jnp.mean(jnp.square(x - m), axis=-1, keepdims=True)
    return (x - m) / jnp.sqrt(v + 1e-5) * g_ref[0, :][None, :] + b_ref[0, :][None, :]


def _k5_body(h_ref, coord_ref, eq_ref, mp_ref, inw_ref, inb_ref,
             lrefs, outw_ref, co_ref, nep_ref):
    hd = H // 4
    mp_src = mp_ref[0] + mp_ref[2]
    mp_dst = mp_ref[1] + mp_ref[3]
    xs = []
    for t in range(NTYPE):
        ht = h_ref[:, t, :]
        e0 = mp_src[:, NTYPE + t] / jnp.maximum(mp_src[:, t], 1.0)
        e1 = mp_dst[:, NTYPE + t] / jnp.maximum(mp_dst[:, t], 1.0)
        x = (jnp.dot(ht.astype(jnp.bfloat16),
                     inw_ref[0:H, :].astype(jnp.bfloat16),
                     preferred_element_type=jnp.float32)
             + e0[:, None] * inw_ref[H, :][None, :]
             + e1[:, None] * inw_ref[H + 1, :][None, :]
             + inb_ref[0, :][None, :])
        xs.append(x)
    for (wq, bq, wk, bk, wv, bv, wo, bo, f1, f1b, f2, f2b,
         g1, b1, g2, b2) in lrefs:
        wqb = wq[...].astype(jnp.bfloat16)
        wkb = wk[...].astype(jnp.bfloat16)
        wvb = wv[...].astype(jnp.bfloat16)
        xb = [x.astype(jnp.bfloat16) for x in xs]
        qs = [jnp.dot(x, wqb, preferred_element_type=jnp.float32) + bq[0, :][None, :]
              for x in xb]
        ks = [jnp.dot(x, wkb, preferred_element_type=jnp.float32) + bk[0, :][None, :]
              for x in xb]
        vs = [jnp.dot(x, wvb, preferred_element_type=jnp.float32) + bv[0, :][None, :]
              for x in xb]
        os_ = []
        for i in range(NTYPE):
            heads = []
            for hh in range(4):
                c0, c1 = hh * hd, (hh + 1) * hd
                a = [jnp.sum(qs[i][:, c0:c1] * ks[j][:, c0:c1], axis=-1) / np.sqrt(hd)
                     for j in range(NTYPE)]
                mx = jnp.maximum(jnp.maximum(a[0], a[1]), jnp.maximum(a[2], a[3]))
                ex = [jnp.exp(aj - mx) for aj in a]
                den = ex[0] + ex[1] + ex[2] + ex[3]
                oh = sum(ex[j][:, None] * vs[j][:, c0:c1] for j in range(NTYPE))
                heads.append(oh / den[:, None])
            os_.append(jnp.concatenate(heads, axis=1))
        wob = wo[...].astype(jnp.bfloat16)
        f1w = f1[...].astype(jnp.bfloat16)
        f2w = f2[...].astype(jnp.bfloat16)
        xs = [_ln(xs[i] + jnp.dot(os_[i].astype(jnp.bfloat16), wob,
                                  preferred_element_type=jnp.float32)
                  + bo[0, :][None, :], g1, b1)
              for i in range(NTYPE)]
        xs = [_ln(x + jnp.dot(_silu(
                  jnp.dot(x.astype(jnp.bfloat16), f1w,
                          preferred_element_type=jnp.float32)
                  + f1b[0, :][None, :]).astype(jnp.bfloat16),
                              f2w, preferred_element_type=jnp.float32)
              + f2b[0, :][None, :], g2, b2)
              for x in xs]
    logits = [jnp.sum(x * outw_ref[0, :][None, :], axis=-1) for x in xs]
    mx = jnp.maximum(jnp.maximum(logits[0], logits[1]),
                     jnp.maximum(logits[2], logits[3]))
    ex = [jnp.exp(l - mx) for l in logits]
    den = ex[0] + ex[1] + ex[2] + ex[3]
    wt = [e / den for e in ex]

    eqs = eq_ref[0] + eq_ref[1]
    outc = []
    for c in range(3):
        nep_c = sum(wt[t] * eqs[:, 4 * t + c] for t in range(NTYPE))
        outc.append(nep_c[:, None])
    nep = jnp.concatenate(outc, axis=1)
    co_ref[...] = coord_ref[...] + nep
    nep_ref[...] = nep


def _k5(h, coord, EQ, MP, ap, n):
    bn = 1000
    grid = n // bn
    full2 = lambda shp: pl.BlockSpec(shp, lambda i: (0, 0))

    layer_inputs = []
    layer_specs = []
    for lp in ap['layers']:
        for nm in ('Wq', 'Wk', 'Wv', 'Wo'):
            layer_inputs += [lp[nm], lp[nm + '_b'].reshape(1, H)]
            layer_specs += [full2((H, H)), full2((1, H))]
        layer_inputs += [lp['F1'], lp['F1_b'].reshape(1, 2 * H),
                         lp['F2'], lp['F2_b'].reshape(1, H),
                         lp['ln1_g'].reshape(1, H), lp['ln1_b'].reshape(1, H),
                         lp['ln2_g'].reshape(1, H), lp['ln2_b'].reshape(1, H)]
        layer_specs += [full2((H, 2 * H)), full2((1, 2 * H)),
                        full2((2 * H, H)), full2((1, H)),
                        full2((1, H)), full2((1, H)),
                        full2((1, H)), full2((1, H))]

    nlayer_refs = 16

    def body(*refs):
        h_ref, coord_ref, eq_ref, mp_ref, inw_ref, inb_ref = refs[:6]
        lref_flat = refs[6:6 + len(layer_specs)]
        outw_ref = refs[6 + len(layer_specs)]
        co_ref, nep_ref = refs[-2:]
        lrefs = [tuple(lref_flat[i * nlayer_refs:(i + 1) * nlayer_refs])
                 for i in range(len(ap['layers']))]
        _k5_body(h_ref, coord_ref, eq_ref, mp_ref, inw_ref, inb_ref,
                 lrefs, outw_ref, co_ref, nep_ref)

    return pl.pallas_call(
        body,
        grid=(grid,),
        in_specs=[
            pl.BlockSpec((bn, NTYPE, H), lambda i: (i, 0, 0)),
            pl.BlockSpec((bn, 3), lambda i: (i, 0)),
            pl.BlockSpec((NC, bn, LANES), lambda i: (0, i, 0)),
            pl.BlockSpec((2 * NC, bn, LANES), lambda i: (0, i, 0)),
            pl.BlockSpec((H + 2, H), lambda i: (0, 0)),
            full2((1, H)),
        ] + layer_specs + [full2((1, H))],
        out_specs=[
            pl.BlockSpec((bn, 3), lambda i: (i, 0)),
            pl.BlockSpec((bn, 3), lambda i: (i, 0)),
        ],
        out_shape=[
            jax.ShapeDtypeStruct((n, 3), jnp.float32),
            jax.ShapeDtypeStruct((n, 3), jnp.float32),
        ],
    )(h, coord, EQ, MP, ap['in_W'], ap['in_b'].reshape(1, H),
      *layer_inputs, ap['out_W'].reshape(1, H))


def kernel(h, coord, edge_index, coord_diff, edge_attr, edge_mask,
           edge_length, N, params):
    del coord_diff
    n = h.shape[0]
    e = edge_index.shape[1]
    src = edge_index[0]
    dst = edge_index[1]
    masks = edge_mask.astype(jnp.float32)
    ea = edge_attr[:, 0]
    el = edge_length[:, 0]
    coordT = coord.T

    names = ('bond', 'angle', 'torsion', 'radius')
    mlps = [params[nm] for nm in names]
    WA = jnp.stack([p[0][0][:H] for p in mlps])
    WB = jnp.stack([p[0][0][H:2 * H] for p in mlps])
    w1ea = jnp.concatenate([p[0][0][2 * H] for p in mlps]).reshape(1, 8 * H)
    b1 = jnp.concatenate([p[0][1] for p in mlps]).reshape(1, 8 * H)
    W2 = jnp.stack([p[1][0] for p in mlps]).astype(jnp.bfloat16)
    b2 = jnp.stack([p[1][1] for p in mlps])
    W3 = jnp.stack([p[2][0] for p in mlps]).astype(jnp.bfloat16)
    b3 = jnp.stack([p[2][1] for p in mlps])
    W4 = jnp.stack([p[3][0][:, 0] for p in mlps])
    b4 = jnp.stack([p[3][1][0] for p in mlps]).reshape(1, NTYPE)

    A, B = _k1(h, WA, WB, n, 1000)
    X1 = _k2(A, B, src, dst, e)
    S = _k3(X1, ea.reshape(1, e), el.reshape(1, e), masks,
            w1ea, b1, W2, b2, W3, b3, W4, b4, e)
    EQ, MP = _k4(src, dst, ea, masks, S, coordT, n, e)
    coord_new, nep = _k5(h, coord, EQ, MP, params['attn'], n)
    return coord_new, nep


# speedup vs baseline: 1.2261x; 1.0272x over previous
import functools
import jax
import jax.numpy as jnp
import numpy as np
from jax import lax
from jax.experimental import pallas as pl
from jax.experimental.pallas import tpu as pltpu
from jax.experimental.pallas import tpu_sc as plsc

H = 128
NTYPE = 4
NC = 2
NS = 16
NW = NC * NS
LANES = 16


def _silu(x):
    return x * jax.nn.sigmoid(x)


def _k1_body(h_ref, wa_ref, wb_ref, a_ref, b_ref):
    for t in range(NTYPE):
        ht = h_ref[:, t, :]
        a_ref[:, t * 2 * H:(t + 1) * 2 * H] = jnp.dot(
            ht, wa_ref[t], preferred_element_type=jnp.float32)
        b_ref[:, t * 2 * H:(t + 1) * 2 * H] = jnp.dot(
            ht, wb_ref[t], preferred_element_type=jnp.float32)


def _k1(h, WA, WB, n, bn):
    grid = n // bn
    return pl.pallas_call(
        _k1_body,
        grid=(grid,),
        in_specs=[
            pl.BlockSpec((bn, NTYPE, H), lambda i: (i, 0, 0)),
            pl.BlockSpec((NTYPE, H, 2 * H), lambda i: (0, 0, 0)),
            pl.BlockSpec((NTYPE, H, 2 * H), lambda i: (0, 0, 0)),
        ],
        out_specs=[
            pl.BlockSpec((bn, 8 * H), lambda i: (i, 0)),
            pl.BlockSpec((bn, 8 * H), lambda i: (i, 0)),
        ],
        out_shape=[
            jax.ShapeDtypeStruct((n, 8 * H), jnp.float32),
            jax.ShapeDtypeStruct((n, 8 * H), jnp.float32),
        ],
    )(h, WA, WB)


def _k2_body(a_hbm, b_hbm, src_hbm, dst_hbm, x1_hbm,
             idx_s, idx_d, buf_a, buf_b0, buf_b1, sem_a, sem_b, wsem):
    E = src_hbm.shape[0]
    ew = E // NW
    C = 40
    D = 8 * H
    nchunk = ew // C
    wid = lax.axis_index("s") * NC + lax.axis_index("c")
    base = wid * ew

    def halfchunk(i, out_buf):
        off = base + i * C
        pltpu.sync_copy(src_hbm.at[pl.ds(off, C)], idx_s)
        pltpu.sync_copy(dst_hbm.at[pl.ds(off, C)], idx_d)
        cp_a = pltpu.async_copy(a_hbm.at[idx_s], buf_a, sem_a)
        cp_b = pltpu.async_copy(b_hbm.at[idx_d], out_buf, sem_b)
        cp_a.wait()
        cp_b.wait()

        def row_add(r, carry2):
            del carry2
            for c in range(D // LANES):
                sl = pl.ds(c * LANES, LANES)
                out_buf[r, sl] = out_buf[r, sl] + buf_a[r, sl]
            return 0

        lax.fori_loop(0, C, row_add, 0)
        return pltpu.async_copy(out_buf, x1_hbm.at[pl.ds(off, C)], wsem)

    def pair(j, carry):
        del carry
        w0 = halfchunk(2 * j, buf_b0)
        w1 = halfchunk(2 * j + 1, buf_b1)
        w0.wait()
        w1.wait()
        return 0

    lax.fori_loop(0, nchunk // 2, pair, 0)


def _k2(A, B, src, dst, e):
    C = 40
    kern = pl.kernel(
        _k2_body,
        out_type=jax.ShapeDtypeStruct((e, 8 * H), jnp.float32),
        mesh=plsc.VectorSubcoreMesh(core_axis_name="c", subcore_axis_name="s"),
        compiler_params=pltpu.CompilerParams(
            needs_layout_passes=False, use_tc_tiling_on_sc=False),
        scratch_types=[
            pltpu.VMEM((C,), jnp.int32),
            pltpu.VMEM((C,), jnp.int32),
            pltpu.VMEM((C, 8 * H), jnp.float32),
            pltpu.VMEM((C, 8 * H), jnp.float32),
            pltpu.VMEM((C, 8 * H), jnp.float32),
            pltpu.SemaphoreType.DMA,
            pltpu.SemaphoreType.DMA,
            pltpu.SemaphoreType.DMA,
        ],
    )
    return kern(A, B, src, dst)


def _k3_body(x1_ref, ea_ref, el_ref, m_ref, w1ea_ref, b1_ref,
             w2_ref, b2_ref, w3_ref, b3_ref, w4_ref, b4_ref, s_ref):
    ea = ea_ref[0, :]
    x = (x1_ref[...]
         + ea[:, None] * w1ea_ref[0, :][None, :] + b1_ref[0, :][None, :])
    x = _silu(x)
    inv_el = 1.0 / el_ref[0, :]
    for t in range(NTYPE):
        xt = x[:, t * 2 * H:(t + 1) * 2 * H].astype(jnp.bfloat16)
        y = _silu(jnp.dot(xt, w2_ref[t], preferred_element_type=jnp.float32)
                  + b2_ref[t][None, :])
        y = _silu(jnp.dot(y.astype(jnp.bfloat16), w3_ref[t],
                          preferred_element_type=jnp.float32)
                  + b3_ref[t][None, :])
        y = jnp.sum(y * w4_ref[t][None, :], axis=-1) + b4_ref[0, t]
        s = jnp.tanh(y) * 10.0
        s_ref[t, :] = s * m_ref[t, :] * inv_el


def _k3(X1, ea2, el2, masks, w1ea2, b12, W2, b2, W3, b3, W4, b4, e):
    be = 512
    grid = e // be
    return pl.pallas_call(
        _k3_body,
        grid=(grid,),
        in_specs=[
            pl.BlockSpec((be, 8 * H), lambda i: (i, 0)),
            pl.BlockSpec((1, be), lambda i: (0, i)),
            pl.BlockSpec((1, be), lambda i: (0, i)),
            pl.BlockSpec((NTYPE, be), lambda i: (0, i)),
            pl.BlockSpec((1, 8 * H), lambda i: (0, 0)),
            pl.BlockSpec((1, 8 * H), lambda i: (0, 0)),
            pl.BlockSpec((NTYPE, 2 * H, H), lambda i: (0, 0, 0)),
            pl.BlockSpec((NTYPE, H), lambda i: (0, 0)),
            pl.BlockSpec((NTYPE, H, H // 2), lambda i: (0, 0, 0)),
            pl.BlockSpec((NTYPE, H // 2), lambda i: (0, 0)),
            pl.BlockSpec((NTYPE, H // 2), lambda i: (0, 0)),
            pl.BlockSpec((1, NTYPE), lambda i: (0, 0)),
        ],
        out_specs=pl.BlockSpec((NTYPE, be), lambda i: (0, i)),
        out_shape=jax.ShapeDtypeStruct((NTYPE, e), jnp.float32),
    )(X1, ea2, el2, masks, w1ea2, b12, W2, b2, W3, b3, W4, b4)


def _k4_body(src_hbm, dst_hbm, ea_hbm, m_hbm, s_hbm, coord_hbm,
             eq_hbm, mp_hbm,
             cx, cy, cz, sbuf, dbuf, eabuf, mbuf, scbuf,
             updA, updB, updM, zrow,
             eq_sh, ms_sh, md_sh, sem):
    E = src_hbm.shape[0]
    N = cx.shape[0]
    ew = E // NW
    C = 80
    nchunk = ew // C
    nv = C // LANES
    cid = lax.axis_index("c")
    sid = lax.axis_index("s")
    wid = sid * NC + cid
    base = wid * ew
    RPT = (N // NS) // 8 * 8
    RLAST = N - (NS - 1) * RPT

    z16 = jnp.zeros((LANES,), jnp.float32)

    def zrow_init(i, carry):
        del carry
        zrow[i, :] = z16
        return 0

    lax.fori_loop(0, RLAST, zrow_init, 0)

    def zbuf_init(i, carry):
        del carry
        updA[i, :] = z16
        updB[i, :] = z16
        updM[i, :] = z16
        return 0

    lax.fori_loop(0, C, zbuf_init, 0)

    r0 = sid * RPT

    @pl.when(sid < NS - 1)
    def _():
        for sh in (eq_sh, ms_sh, md_sh):
            pltpu.sync_copy(zrow.at[pl.ds(0, RPT)], sh.at[pl.ds(r0, RPT)])

    @pl.when(sid == NS - 1)
    def _():
        for sh in (eq_sh, ms_sh, md_sh):
            pltpu.sync_copy(zrow, sh.at[pl.ds(r0, RLAST)])

    pltpu.sync_copy(coord_hbm.at[pl.ds(0, N)], cx)
    pltpu.sync_copy(coord_hbm.at[pl.ds(N, N)], cy)
    pltpu.sync_copy(coord_hbm.at[pl.ds(2 * N, N)], cz)
    plsc.subcore_barrier()

    def chunk(i, carry):
        del carry
        off = base + i * C
        pltpu.sync_copy(src_hbm.at[pl.ds(off, C)], sbuf)
        pltpu.sync_copy(dst_hbm.at[pl.ds(off, C)], dbuf)
        pltpu.sync_copy(ea_hbm.at[pl.ds(off, C)], eabuf)
        for t in range(NTYPE):
            pltpu.sync_copy(m_hbm.at[pl.ds(t * E + off, C)],
                            mbuf.at[pl.ds(t * C, C)])
            pltpu.sync_copy(s_hbm.at[pl.ds(t * E + off, C)],
                            scbuf.at[pl.ds(t * C, C)])
        for v in range(nv):
            sl = pl.ds(v * LANES, LANES)
            s16 = sbuf[sl]
            d16 = dbuf[sl]
            ea16 = eabuf[sl]
            ddx = plsc.load_gather(cx, [s16]) - plsc.load_gather(cx, [d16])
            ddy = plsc.load_gather(cy, [s16]) - plsc.load_gather(cy, [d16])
            ddz = plsc.load_gather(cz, [s16]) - plsc.load_gather(cz, [d16])
            row16 = lax.iota(jnp.int32, LANES) + (v * LANES)
            for t in range(NTYPE):
                st = scbuf[pl.ds(t * C + v * LANES, LANES)]
                ct = jnp.full((LANES,), 4 * t, jnp.int32)
                vx = ddx * st
                vy = ddy * st
                vz = ddz * st
                plsc.store_scatter(updA, [row16, ct], vx)
                plsc.store_scatter(updA, [row16, ct + 1], vy)
                plsc.store_scatter(updA, [row16, ct + 2], vz)
                plsc.store_scatter(updB, [row16, ct], -vx)
                plsc.store_scatter(updB, [row16, ct + 1], -vy)
                plsc.store_scatter(updB, [row16, ct + 2], -vz)
                wt = mbuf[pl.ds(t * C + v * LANES, LANES)]
                ctm = jnp.full((LANES,), t, jnp.int32)
                plsc.store_scatter(updM, [row16, ctm], wt)
                plsc.store_scatter(updM, [row16, ctm + NTYPE], wt * ea16)
        pltpu.sync_copy(updA, eq_sh.at[sbuf], add=True)
        pltpu.sync_copy(updB, eq_sh.at[dbuf], add=True)
        pltpu.sync_copy(updM, ms_sh.at[sbuf], add=True)
        pltpu.sync_copy(updM, md_sh.at[dbuf], add=True)
        return 0

    lax.fori_loop(0, nchunk, chunk, 0)
    plsc.subcore_barrier()

    pairs = ((eq_sh, eq_hbm, cid * N), (ms_sh, mp_hbm, 2 * cid * N),
             (md_sh, mp_hbm, (2 * cid + 1) * N))

    @pl.when(sid < NS - 1)
    def _():
        for sh, ob, o0 in pairs:
            pltpu.sync_copy(sh.at[pl.ds(r0, RPT)], ob.at[pl.ds(o0 + r0, RPT)])

    @pl.when(sid == NS - 1)
    def _():
        for sh, ob, o0 in pairs:
            pltpu.sync_copy(sh.at[pl.ds(r0, RLAST)],
                            ob.at[pl.ds(o0 + r0, RLAST)])


def _k4(src, dst, ea, masks, S, coordT, n, e):
    C = 80
    rows_per_tile = n - (NS - 1) * ((n // NS) // 8 * 8)
    kern = pl.kernel(
        _k4_body,
        out_type=[
            jax.ShapeDtypeStruct((NC * n, LANES), jnp.float32),
            jax.ShapeDtypeStruct((2 * NC * n, LANES), jnp.float32),
        ],
        mesh=plsc.VectorSubcoreMesh(core_axis_name="c", subcore_axis_name="s"),
        compiler_params=pltpu.CompilerParams(
            needs_layout_passes=False, use_tc_tiling_on_sc=False),
        scratch_types=[
            pltpu.VMEM((n,), jnp.float32),
            pltpu.VMEM((n,), jnp.float32),
            pltpu.VMEM((n,), jnp.float32),
            pltpu.VMEM((C,), jnp.int32),
            pltpu.VMEM((C,), jnp.int32),
            pltpu.VMEM((C,), jnp.float32),
            pltpu.VMEM((NTYPE * C,), jnp.float32),
            pltpu.VMEM((NTYPE * C,), jnp.float32),
            pltpu.VMEM((C, LANES), jnp.float32),
            pltpu.VMEM((C, LANES), jnp.float32),
            pltpu.VMEM((C, LANES), jnp.float32),
            pltpu.VMEM((rows_per_tile, LANES), jnp.float32),
            pltpu.VMEM_SHARED((n, LANES), jnp.float32),
            pltpu.VMEM_SHARED((n, LANES), jnp.float32),
            pltpu.VMEM_SHARED((n, LANES), jnp.float32),
            pltpu.SemaphoreType.DMA,
        ],
    )
    eq_f, mp_f = kern(src, dst, ea, masks.reshape(-1), S.reshape(-1),
                      coordT.reshape(-1))
    return (eq_f.reshape(NC, n, LANES), mp_f.reshape(2 * NC, n, LANES))


def _ln(x, g_ref, b_ref):
    m = jnp.mean(x, axis=-1, keepdims=True)
    v = jnp.mean(jnp.square(x - m), axis=-1, keepdims=True)
    return (x - m) / jnp.sqrt(v + 1e-5) * g_ref[0, :][None, :] + b_ref[0, :][None, :]


def _k5_body(h_ref, coord_ref, eq_ref, mp_ref, inw_ref, inb_ref,
             lrefs, outw_ref, co_ref, nep_ref):
    hd = H // 4
    mp_src = mp_ref[0] + mp_ref[2]
    mp_dst = mp_ref[1] + mp_ref[3]
    xs = []
    for t in range(NTYPE):
        ht = h_ref[:, t, :]
        e0 = mp_src[:, NTYPE + t] / jnp.maximum(mp_src[:, t], 1.0)
        e1 = mp_dst[:, NTYPE + t] / jnp.maximum(mp_dst[:, t], 1.0)
        x = (jnp.dot(ht.astype(jnp.bfloat16),
                     inw_ref[0:H, :].astype(jnp.bfloat16),
                     preferred_element_type=jnp.float32)
             + e0[:, None] * inw_ref[H, :][None, :]
             + e1[:, None] * inw_ref[H + 1, :][None, :]
             + inb_ref[0, :][None, :])
        xs.append(x)
    for (wq, bq, wk, bk, wv, bv, wo, bo, f1, f1b, f2, f2b,
         g1, b1, g2, b2) in lrefs:
        wqb = wq[...].astype(jnp.bfloat16)
        wkb = wk[...].astype(jnp.bfloat16)
        wvb = wv[...].astype(jnp.bfloat16)
        xb = [x.astype(jnp.bfloat16) for x in xs]
        qs = [jnp.dot(x, wqb, preferred_element_type=jnp.float32) + bq[0, :][None, :]
              for x in xb]
        ks = [jnp.dot(x, wkb, preferred_element_type=jnp.float32) + bk[0, :][None, :]
              for x in xb]
        vs = [jnp.dot(x, wvb, preferred_element_type=jnp.float32) + bv[0, :][None, :]
              for x in xb]
        os_ = []
        for i in range(NTYPE):
            heads = []
            for hh in range(4):
                c0, c1 = hh * hd, (hh + 1) * hd
                a = [jnp.sum(qs[i][:, c0:c1] * ks[j][:, c0:c1], axis=-1) / np.sqrt(hd)
                     for j in range(NTYPE)]
                mx = jnp.maximum(jnp.maximum(a[0], a[1]), jnp.maximum(a[2], a[3]))
                ex = [jnp.exp(aj - mx) for aj in a]
                den = ex[0] + ex[1] + ex[2] + ex[3]
                oh = sum(ex[j][:, None] * vs[j][:, c0:c1] for j in range(NTYPE))
                heads.append(oh / den[:, None])
            os_.append(jnp.concatenate(heads, axis=1))
        wob = wo[...].astype(jnp.bfloat16)
        f1w = f1[...].astype(jnp.bfloat16)
        f2w = f2[...].astype(jnp.bfloat16)
        xs = [_ln(xs[i] + jnp.dot(os_[i].astype(jnp.bfloat16), wob,
                                  preferred_element_type=jnp.float32)
                  + bo[0, :][None, :], g1, b1)
              for i in range(NTYPE)]
        xs = [_ln(x + jnp.dot(_silu(
                  jnp.dot(x.astype(jnp.bfloat16), f1w,
                          preferred_element_type=jnp.float32)
                  + f1b[0, :][None, :]).astype(jnp.bfloat16),
                              f2w, preferred_element_type=jnp.float32)
              + f2b[0, :][None, :], g2, b2)
              for x in xs]
    logits = [jnp.sum(x * outw_ref[0, :][None, :], axis=-1) for x in xs]
    mx = jnp.maximum(jnp.maximum(logits[0], logits[1]),
                     jnp.maximum(logits[2], logits[3]))
    ex = [jnp.exp(l - mx) for l in logits]
    den = ex[0] + ex[1] + ex[2] + ex[3]
    wt = [e / den for e in ex]

    eqs = eq_ref[0] + eq_ref[1]
    outc = []
    for c in range(3):
        nep_c = sum(wt[t] * eqs[:, 4 * t + c] for t in range(NTYPE))
        outc.append(nep_c[:, None])
    nep = jnp.concatenate(outc, axis=1)
    co_ref[...] = coord_ref[...] + nep
    nep_ref[...] = nep


def _k5(h, coord, EQ, MP, ap, n):
    bn = 1000
    grid = n // bn
    full2 = lambda shp: pl.BlockSpec(shp, lambda i: (0, 0))

    layer_inputs = []
    layer_specs = []
    for lp in ap['layers']:
        for nm in ('Wq', 'Wk', 'Wv', 'Wo'):
            layer_inputs += [lp[nm], lp[nm + '_b'].reshape(1, H)]
            layer_specs += [full2((H, H)), full2((1, H))]
        layer_inputs += [lp['F1'], lp['F1_b'].reshape(1, 2 * H),
                         lp['F2'], lp['F2_b'].reshape(1, H),
                         lp['ln1_g'].reshape(1, H), lp['ln1_b'].reshape(1, H),
                         lp['ln2_g'].reshape(1, H), lp['ln2_b'].reshape(1, H)]
        layer_specs += [full2((H, 2 * H)), full2((1, 2 * H)),
                        full2((2 * H, H)), full2((1, H)),
                        full2((1, H)), full2((1, H)),
                        full2((1, H)), full2((1, H))]

    nlayer_refs = 16

    def body(*refs):
        h_ref, coord_ref, eq_ref, mp_ref, inw_ref, inb_ref = refs[:6]
        lref_flat = refs[6:6 + len(layer_specs)]
        outw_ref = refs[6 + len(layer_specs)]
        co_ref, nep_ref = refs[-2:]
        lrefs = [tuple(lref_flat[i * nlayer_refs:(i + 1) * nlayer_refs])
                 for i in range(len(ap['layers']))]
        _k5_body(h_ref, coord_ref, eq_ref, mp_ref, inw_ref, inb_ref,
                 lrefs, outw_ref, co_ref, nep_ref)

    return pl.pallas_call(
        body,
        grid=(grid,),
        in_specs=[
            pl.BlockSpec((bn, NTYPE, H), lambda i: (i, 0, 0)),
            pl.BlockSpec((bn, 3), lambda i: (i, 0)),
            pl.BlockSpec((NC, bn, LANES), lambda i: (0, i, 0)),
            pl.BlockSpec((2 * NC, bn, LANES), lambda i: (0, i, 0)),
            pl.BlockSpec((H + 2, H), lambda i: (0, 0)),
            full2((1, H)),
        ] + layer_specs + [full2((1, H))],
        out_specs=[
            pl.BlockSpec((bn, 3), lambda i: (i, 0)),
            pl.BlockSpec((bn, 3), lambda i: (i, 0)),
        ],
        out_shape=[
            jax.ShapeDtypeStruct((n, 3), jnp.float32),
            jax.ShapeDtypeStruct((n, 3), jnp.float32),
        ],
    )(h, coord, EQ, MP, ap['in_W'], ap['in_b'].reshape(1, H),
      *layer_inputs, ap['out_W'].reshape(1, H))


def kernel(h, coord, edge_index, coord_diff, edge_attr, edge_mask,
           edge_length, N, params):
    del coord_diff
    n = h.shape[0]
    e = edge_index.shape[1]
    src = edge_index[0]
    dst = edge_index[1]
    masks = edge_mask.astype(jnp.float32)
    ea = edge_attr[:, 0]
    el = edge_length[:, 0]
    coordT = coord.T

    names = ('bond', 'angle', 'torsion', 'radius')
    mlps = [params[nm] for nm in names]
    WA = jnp.stack([p[0][0][:H] for p in mlps])
    WB = jnp.stack([p[0][0][H:2 * H] for p in mlps])
    w1ea = jnp.concatenate([p[0][0][2 * H] for p in mlps]).reshape(1, 8 * H)
    b1 = jnp.concatenate([p[0][1] for p in mlps]).reshape(1, 8 * H)
    W2 = jnp.stack([p[1][0] for p in mlps]).astype(jnp.bfloat16)
    b2 = jnp.stack([p[1][1] for p in mlps])
    W3 = jnp.stack([p[2][0] for p in mlps]).astype(jnp.bfloat16)
    b3 = jnp.stack([p[2][1] for p in mlps])
    W4 = jnp.stack([p[3][0][:, 0] for p in mlps])
    b4 = jnp.stack([p[3][1][0] for p in mlps]).reshape(1, NTYPE)

    A, B = _k1(h, WA, WB, n, 1000)
    X1 = _k2(A, B, src, dst, e)
    S = _k3(X1, ea.reshape(1, e), el.reshape(1, e), masks,
            w1ea, b1, W2, b2, W3, b3, W4, b4, e)
    EQ, MP = _k4(src, dst, ea, masks, S, coordT, n, e)
    coord_new, nep = _k5(h, coord, EQ, MP, params['attn'], n)
    return coord_new, nep


# speedup vs baseline: 1.3147x; 1.0723x over previous
import functools
import jax
import jax.numpy as jnp
import numpy as np
from jax import lax
from jax.experimental import pallas as pl
from jax.experimental.pallas import tpu as pltpu
from jax.experimental.pallas import tpu_sc as plsc

H = 128
NTYPE = 4
NC = 2
NS = 16
NW = NC * NS
LANES = 16


def _silu(x):
    return x * jax.nn.sigmoid(x)


def _k1_body(h_ref, wa_ref, wb_ref, a_ref, b_ref):
    for t in range(NTYPE):
        ht = h_ref[:, t, :]
        a_ref[:, t * 2 * H:(t + 1) * 2 * H] = jnp.dot(
            ht, wa_ref[t], preferred_element_type=jnp.float32)
        b_ref[:, t * 2 * H:(t + 1) * 2 * H] = jnp.dot(
            ht, wb_ref[t], preferred_element_type=jnp.float32)


def _k1(h, WA, WB, n, bn):
    grid = n // bn
    return pl.pallas_call(
        _k1_body,
        grid=(grid,),
        in_specs=[
            pl.BlockSpec((bn, NTYPE, H), lambda i: (i, 0, 0)),
            pl.BlockSpec((NTYPE, H, 2 * H), lambda i: (0, 0, 0)),
            pl.BlockSpec((NTYPE, H, 2 * H), lambda i: (0, 0, 0)),
        ],
        out_specs=[
            pl.BlockSpec((bn, 8 * H), lambda i: (i, 0)),
            pl.BlockSpec((bn, 8 * H), lambda i: (i, 0)),
        ],
        out_shape=[
            jax.ShapeDtypeStruct((n, 8 * H), jnp.float32),
            jax.ShapeDtypeStruct((n, 8 * H), jnp.float32),
        ],
    )(h, WA, WB)


def _k2_body(a_hbm, b_hbm, src_hbm, dst_hbm, x1_hbm,
             idx_s, idx_d, buf_a, buf_b0, buf_b1, sem_a, sem_b, wsem):
    E = src_hbm.shape[0]
    ew = E // NW
    C = 40
    D = 8 * H
    nchunk = ew // C
    wid = lax.axis_index("s") * NC + lax.axis_index("c")
    base = wid * ew

    def halfchunk(i, out_buf):
        off = base + i * C
        pltpu.sync_copy(src_hbm.at[pl.ds(off, C)], idx_s)
        pltpu.sync_copy(dst_hbm.at[pl.ds(off, C)], idx_d)
        cp_a = pltpu.async_copy(a_hbm.at[idx_s], buf_a, sem_a)
        cp_b = pltpu.async_copy(b_hbm.at[idx_d], out_buf, sem_b)
        cp_a.wait()
        cp_b.wait()

        def row_add(r, carry2):
            del carry2
            for c in range(D // LANES):
                sl = pl.ds(c * LANES, LANES)
                out_buf[r, sl] = out_buf[r, sl] + buf_a[r, sl]
            return 0

        lax.fori_loop(0, C, row_add, 0)
        return pltpu.async_copy(out_buf, x1_hbm.at[pl.ds(off, C)], wsem)

    def pair(j, carry):
        del carry
        w0 = halfchunk(2 * j, buf_b0)
        w1 = halfchunk(2 * j + 1, buf_b1)
        w0.wait()
        w1.wait()
        return 0

    lax.fori_loop(0, nchunk // 2, pair, 0)


def _k2(A, B, src, dst, e):
    C = 40
    kern = pl.kernel(
        _k2_body,
        out_type=jax.ShapeDtypeStruct((e, 8 * H), jnp.float32),
        mesh=plsc.VectorSubcoreMesh(core_axis_name="c", subcore_axis_name="s"),
        compiler_params=pltpu.CompilerParams(
            needs_layout_passes=False, use_tc_tiling_on_sc=False),
        scratch_types=[
            pltpu.VMEM((C,), jnp.int32),
            pltpu.VMEM((C,), jnp.int32),
            pltpu.VMEM((C, 8 * H), jnp.float32),
            pltpu.VMEM((C, 8 * H), jnp.float32),
            pltpu.VMEM((C, 8 * H), jnp.float32),
            pltpu.SemaphoreType.DMA,
            pltpu.SemaphoreType.DMA,
            pltpu.SemaphoreType.DMA,
        ],
    )
    return kern(A, B, src, dst)


def _k3_body(x1_ref, ea_ref, el_ref, m_ref, w1ea_ref, b1_ref,
             w2_ref, b2_ref, w3_ref, b3_ref, w4_ref, b4_ref, s_ref):
    ea = ea_ref[0, :]
    x = (x1_ref[...]
         + ea[:, None] * w1ea_ref[0, :][None, :] + b1_ref[0, :][None, :])
    x = _silu(x)
    inv_el = 1.0 / el_ref[0, :]
    for t in range(NTYPE):
        xt = x[:, t * 2 * H:(t + 1) * 2 * H].astype(jnp.bfloat16)
        y = _silu(jnp.dot(xt, w2_ref[t], preferred_element_type=jnp.float32)
                  + b2_ref[t][None, :])
        y = _silu(jnp.dot(y.astype(jnp.bfloat16), w3_ref[t],
                          preferred_element_type=jnp.float32)
                  + b3_ref[t][None, :])
        y = jnp.sum(y * w4_ref[t][None, :], axis=-1) + b4_ref[0, t]
        s = jnp.tanh(y) * 10.0
        s_ref[t, :] = s * m_ref[t, :] * inv_el


def _k3(X1, ea2, el2, masks, w1ea2, b12, W2, b2, W3, b3, W4, b4, e):
    be = 512
    grid = e // be
    return pl.pallas_call(
        _k3_body,
        grid=(grid,),
        in_specs=[
            pl.BlockSpec((be, 8 * H), lambda i: (i, 0)),
            pl.BlockSpec((1, be), lambda i: (0, i)),
            pl.BlockSpec((1, be), lambda i: (0, i)),
            pl.BlockSpec((NTYPE, be), lambda i: (0, i)),
            pl.BlockSpec((1, 8 * H), lambda i: (0, 0)),
            pl.BlockSpec((1, 8 * H), lambda i: (0, 0)),
            pl.BlockSpec((NTYPE, 2 * H, H), lambda i: (0, 0, 0)),
            pl.BlockSpec((NTYPE, H), lambda i: (0, 0)),
            pl.BlockSpec((NTYPE, H, H // 2), lambda i: (0, 0, 0)),
            pl.BlockSpec((NTYPE, H // 2), lambda i: (0, 0)),
            pl.BlockSpec((NTYPE, H // 2), lambda i: (0, 0)),
            pl.BlockSpec((1, NTYPE), lambda i: (0, 0)),
        ],
        out_specs=pl.BlockSpec((NTYPE, be), lambda i: (0, i)),
        out_shape=jax.ShapeDtypeStruct((NTYPE, e), jnp.float32),
    )(X1, ea2, el2, masks, w1ea2, b12, W2, b2, W3, b3, W4, b4)


def _k4_body(src_hbm, dst_hbm, ea_hbm, m_hbm, s_hbm, coord_hbm,
             eq_hbm, mp_hbm,
             cx, cy, cz, sbuf, dbuf, eabuf, mbuf, scbuf,
             updA, updB, updM, zrow,
             eq_sh, ms_sh, md_sh, sem):
    E = src_hbm.shape[0]
    N = cx.shape[0]
    ew = E // NW
    C = 400
    nchunk = ew // C
    nv = C // LANES
    cid = lax.axis_index("c")
    sid = lax.axis_index("s")
    wid = sid * NC + cid
    base = wid * ew
    RPT = (N // NS) // 8 * 8
    RLAST = N - (NS - 1) * RPT

    z16 = jnp.zeros((LANES,), jnp.float32)

    def zrow_init(i, carry):
        del carry
        zrow[i, :] = z16
        return 0

    lax.fori_loop(0, RLAST, zrow_init, 0)

    def zbuf_init(i, carry):
        del carry
        updA[i, :] = z16
        updB[i, :] = z16
        updM[i, :] = z16
        return 0

    lax.fori_loop(0, C, zbuf_init, 0)

    r0 = sid * RPT

    @pl.when(sid < NS - 1)
    def _():
        for sh in (eq_sh, ms_sh, md_sh):
            pltpu.sync_copy(zrow.at[pl.ds(0, RPT)], sh.at[pl.ds(r0, RPT)])

    @pl.when(sid == NS - 1)
    def _():
        for sh in (eq_sh, ms_sh, md_sh):
            pltpu.sync_copy(zrow, sh.at[pl.ds(r0, RLAST)])

    pltpu.sync_copy(coord_hbm.at[pl.ds(0, N)], cx)
    pltpu.sync_copy(coord_hbm.at[pl.ds(N, N)], cy)
    pltpu.sync_copy(coord_hbm.at[pl.ds(2 * N, N)], cz)
    plsc.subcore_barrier()

    def chunk(i, carry):
        del carry
        off = base + i * C
        pltpu.sync_copy(src_hbm.at[pl.ds(off, C)], sbuf)
        pltpu.sync_copy(dst_hbm.at[pl.ds(off, C)], dbuf)
        pltpu.sync_copy(ea_hbm.at[pl.ds(off, C)], eabuf)
        for t in range(NTYPE):
            pltpu.sync_copy(m_hbm.at[pl.ds(t * E + off, C)],
                            mbuf.at[pl.ds(t * C, C)])
            pltpu.sync_copy(s_hbm.at[pl.ds(t * E + off, C)],
                            scbuf.at[pl.ds(t * C, C)])
        for v in range(nv):
            sl = pl.ds(v * LANES, LANES)
            s16 = sbuf[sl]
            d16 = dbuf[sl]
            ea16 = eabuf[sl]
            ddx = plsc.load_gather(cx, [s16]) - plsc.load_gather(cx, [d16])
            ddy = plsc.load_gather(cy, [s16]) - plsc.load_gather(cy, [d16])
            ddz = plsc.load_gather(cz, [s16]) - plsc.load_gather(cz, [d16])
            row16 = lax.iota(jnp.int32, LANES) + (v * LANES)
            for t in range(NTYPE):
                st = scbuf[pl.ds(t * C + v * LANES, LANES)]
                ct = jnp.full((LANES,), 4 * t, jnp.int32)
                vx = ddx * st
                vy = ddy * st
                vz = ddz * st
                plsc.store_scatter(updA, [row16, ct], vx)
                plsc.store_scatter(updA, [row16, ct + 1], vy)
                plsc.store_scatter(updA, [row16, ct + 2], vz)
                plsc.store_scatter(updB, [row16, ct], -vx)
                plsc.store_scatter(updB, [row16, ct + 1], -vy)
                plsc.store_scatter(updB, [row16, ct + 2], -vz)
                wt = mbuf[pl.ds(t * C + v * LANES, LANES)]
                ctm = jnp.full((LANES,), t, jnp.int32)
                plsc.store_scatter(updM, [row16, ctm], wt)
                plsc.store_scatter(updM, [row16, ctm + NTYPE], wt * ea16)
        pltpu.sync_copy(updA, eq_sh.at[sbuf], add=True)
        pltpu.sync_copy(updB, eq_sh.at[dbuf], add=True)
        pltpu.sync_copy(updM, ms_sh.at[sbuf], add=True)
        pltpu.sync_copy(updM, md_sh.at[dbuf], add=True)
        return 0

    lax.fori_loop(0, nchunk, chunk, 0)
    plsc.subcore_barrier()

    pairs = ((eq_sh, eq_hbm, cid * N), (ms_sh, mp_hbm, 2 * cid * N),
             (md_sh, mp_hbm, (2 * cid + 1) * N))

    @pl.when(sid < NS - 1)
    def _():
        for sh, ob, o0 in pairs:
            pltpu.sync_copy(sh.at[pl.ds(r0, RPT)], ob.at[pl.ds(o0 + r0, RPT)])

    @pl.when(sid == NS - 1)
    def _():
        for sh, ob, o0 in pairs:
            pltpu.sync_copy(sh.at[pl.ds(r0, RLAST)],
                            ob.at[pl.ds(o0 + r0, RLAST)])


def _k4(src, dst, ea, masks, S, coordT, n, e):
    C = 400
    rows_per_tile = n - (NS - 1) * ((n // NS) // 8 * 8)
    kern = pl.kernel(
        _k4_body,
        out_type=[
            jax.ShapeDtypeStruct((NC * n, LANES), jnp.float32),
            jax.ShapeDtypeStruct((2 * NC * n, LANES), jnp.float32),
        ],
        mesh=plsc.VectorSubcoreMesh(core_axis_name="c", subcore_axis_name="s"),
        compiler_params=pltpu.CompilerParams(
            needs_layout_passes=False, use_tc_tiling_on_sc=False),
        scratch_types=[
            pltpu.VMEM((n,), jnp.float32),
            pltpu.VMEM((n,), jnp.float32),
            pltpu.VMEM((n,), jnp.float32),
            pltpu.VMEM((C,), jnp.int32),
            pltpu.VMEM((C,), jnp.int32),
            pltpu.VMEM((C,), jnp.float32),
            pltpu.VMEM((NTYPE * C,), jnp.float32),
            pltpu.VMEM((NTYPE * C,), jnp.float32),
            pltpu.VMEM((C, LANES), jnp.float32),
            pltpu.VMEM((C, LANES), jnp.float32),
            pltpu.VMEM((C, LANES), jnp.float32),
            pltpu.VMEM((rows_per_tile, LANES), jnp.float32),
            pltpu.VMEM_SHARED((n, LANES), jnp.float32),
            pltpu.VMEM_SHARED((n, LANES), jnp.float32),
            pltpu.VMEM_SHARED((n, LANES), jnp.float32),
            pltpu.SemaphoreType.DMA,
        ],
    )
    eq_f, mp_f = kern(src, dst, ea, masks.reshape(-1), S.reshape(-1),
                      coordT.reshape(-1))
    return (eq_f.reshape(NC, n, LANES), mp_f.reshape(2 * NC, n, LANES))


def _ln(x, g_ref, b_ref):
    m = jnp.mean(x, axis=-1, keepdims=True)
    v = jnp.mean(jnp.square(x - m), axis=-1, keepdims=True)
    return (x - m) / jnp.sqrt(v + 1e-5) * g_ref[0, :][None, :] + b_ref[0, :][None, :]


def _k5_body(h_ref, coord_ref, eq_ref, mp_ref, inw_ref, inb_ref,
             lrefs, outw_ref, co_ref, nep_ref):
    hd = H // 4
    mp_src = mp_ref[0] + mp_ref[2]
    mp_dst = mp_ref[1] + mp_ref[3]
    xs = []
    for t in range(NTYPE):
        ht = h_ref[:, t, :]
        e0 = mp_src[:, NTYPE + t] / jnp.maximum(mp_src[:, t], 1.0)
        e1 = mp_dst[:, NTYPE + t] / jnp.maximum(mp_dst[:, t], 1.0)
        x = (jnp.dot(ht.astype(jnp.bfloat16),
                     inw_ref[0:H, :].astype(jnp.bfloat16),
                     preferred_element_type=jnp.float32)
             + e0[:, None] * inw_ref[H, :][None, :]
             + e1[:, None] * inw_ref[H + 1, :][None, :]
             + inb_ref[0, :][None, :])
        xs.append(x)
    for (wq, bq, wk, bk, wv, bv, wo, bo, f1, f1b, f2, f2b,
         g1, b1, g2, b2) in lrefs:
        wqb = wq[...].astype(jnp.bfloat16)
        wkb = wk[...].astype(jnp.bfloat16)
        wvb = wv[...].astype(jnp.bfloat16)
        xb = [x.astype(jnp.bfloat16) for x in xs]
        qs = [jnp.dot(x, wqb, preferred_element_type=jnp.float32) + bq[0, :][None, :]
              for x in xb]
        ks = [jnp.dot(x, wkb, preferred_element_type=jnp.float32) + bk[0, :][None, :]
              for x in xb]
        vs = [jnp.dot(x, wvb, preferred_element_type=jnp.float32) + bv[0, :][None, :]
              for x in xb]
        os_ = []
        for i in range(NTYPE):
            heads = []
            for hh in range(4):
                c0, c1 = hh * hd, (hh + 1) * hd
                a = [jnp.sum(qs[i][:, c0:c1] * ks[j][:, c0:c1], axis=-1) / np.sqrt(hd)
                     for j in range(NTYPE)]
                mx = jnp.maximum(jnp.maximum(a[0], a[1]), jnp.maximum(a[2], a[3]))
                ex = [jnp.exp(aj - mx) for aj in a]
                den = ex[0] + ex[1] + ex[2] + ex[3]
                oh = sum(ex[j][:, None] * vs[j][:, c0:c1] for j in range(NTYPE))
                heads.append(oh / den[:, None])
            os_.append(jnp.concatenate(heads, axis=1))
        wob = wo[...].astype(jnp.bfloat16)
        f1w = f1[...].astype(jnp.bfloat16)
        f2w = f2[...].astype(jnp.bfloat16)
        xs = [_ln(xs[i] + jnp.dot(os_[i].astype(jnp.bfloat16), wob,
                                  preferred_element_type=jnp.float32)
                  + bo[0, :][None, :], g1, b1)
              for i in range(NTYPE)]
        xs = [_ln(x + jnp.dot(_silu(
                  jnp.dot(x.astype(jnp.bfloat16), f1w,
                          preferred_element_type=jnp.float32)
                  + f1b[0, :][None, :]).astype(jnp.bfloat16),
                              f2w, preferred_element_type=jnp.float32)
              + f2b[0, :][None, :], g2, b2)
              for x in xs]
    logits = [jnp.sum(x * outw_ref[0, :][None, :], axis=-1) for x in xs]
    mx = jnp.maximum(jnp.maximum(logits[0], logits[1]),
                     jnp.maximum(logits[2], logits[3]))
    ex = [jnp.exp(l - mx) for l in logits]
    den = ex[0] + ex[1] + ex[2] + ex[3]
    wt = [e / den for e in ex]

    eqs = eq_ref[0] + eq_ref[1]
    outc = []
    for c in range(3):
        nep_c = sum(wt[t] * eqs[:, 4 * t + c] for t in range(NTYPE))
        outc.append(nep_c[:, None])
    nep = jnp.concatenate(outc, axis=1)
    co_ref[...] = coord_ref[...] + nep
    nep_ref[...] = nep


def _k5(h, coord, EQ, MP, ap, n):
    bn = 1000
    grid = n // bn
    full2 = lambda shp: pl.BlockSpec(shp, lambda i: (0, 0))

    layer_inputs = []
    layer_specs = []
    for lp in ap['layers']:
        for nm in ('Wq', 'Wk', 'Wv', 'Wo'):
            layer_inputs += [lp[nm], lp[nm + '_b'].reshape(1, H)]
            layer_specs += [full2((H, H)), full2((1, H))]
        layer_inputs += [lp['F1'], lp['F1_b'].reshape(1, 2 * H),
                         lp['F2'], lp['F2_b'].reshape(1, H),
                         lp['ln1_g'].reshape(1, H), lp['ln1_b'].reshape(1, H),
                         lp['ln2_g'].reshape(1, H), lp['ln2_b'].reshape(1, H)]
        layer_specs += [full2((H, 2 * H)), full2((1, 2 * H)),
                        full2((2 * H, H)), full2((1, H)),
                        full2((1, H)), full2((1, H)),
                        full2((1, H)), full2((1, H))]

    nlayer_refs = 16

    def body(*refs):
        h_ref, coord_ref, eq_ref, mp_ref, inw_ref, inb_ref = refs[:6]
        lref_flat = refs[6:6 + len(layer_specs)]
        outw_ref = refs[6 + len(layer_specs)]
        co_ref, nep_ref = refs[-2:]
        lrefs = [tuple(lref_flat[i * nlayer_refs:(i + 1) * nlayer_refs])
                 for i in range(len(ap['layers']))]
        _k5_body(h_ref, coord_ref, eq_ref, mp_ref, inw_ref, inb_ref,
                 lrefs, outw_ref, co_ref, nep_ref)

    return pl.pallas_call(
        body,
        grid=(grid,),
        in_specs=[
            pl.BlockSpec((bn, NTYPE, H), lambda i: (i, 0, 0)),
            pl.BlockSpec((bn, 3), lambda i: (i, 0)),
            pl.BlockSpec((NC, bn, LANES), lambda i: (0, i, 0)),
            pl.BlockSpec((2 * NC, bn, LANES), lambda i: (0, i, 0)),
            pl.BlockSpec((H + 2, H), lambda i: (0, 0)),
            full2((1, H)),
        ] + layer_specs + [full2((1, H))],
        out_specs=[
            pl.BlockSpec((bn, 3), lambda i: (i, 0)),
            pl.BlockSpec((bn, 3), lambda i: (i, 0)),
        ],
        out_shape=[
            jax.ShapeDtypeStruct((n, 3), jnp.float32),
            jax.ShapeDtypeStruct((n, 3), jnp.float32),
        ],
    )(h, coord, EQ, MP, ap['in_W'], ap['in_b'].reshape(1, H),
      *layer_inputs, ap['out_W'].reshape(1, H))


def kernel(h, coord, edge_index, coord_diff, edge_attr, edge_mask,
           edge_length, N, params):
    del coord_diff
    n = h.shape[0]
    e = edge_index.shape[1]
    src = edge_index[0]
    dst = edge_index[1]
    masks = edge_mask.astype(jnp.float32)
    ea = edge_attr[:, 0]
    el = edge_length[:, 0]
    coordT = coord.T

    names = ('bond', 'angle', 'torsion', 'radius')
    mlps = [params[nm] for nm in names]
    WA = jnp.stack([p[0][0][:H] for p in mlps])
    WB = jnp.stack([p[0][0][H:2 * H] for p in mlps])
    w1ea = jnp.concatenate([p[0][0][2 * H] for p in mlps]).reshape(1, 8 * H)
    b1 = jnp.concatenate([p[0][1] for p in mlps]).reshape(1, 8 * H)
    W2 = jnp.stack([p[1][0] for p in mlps]).astype(jnp.bfloat16)
    b2 = jnp.stack([p[1][1] for p in mlps])
    W3 = jnp.stack([p[2][0] for p in mlps]).astype(jnp.bfloat16)
    b3 = jnp.stack([p[2][1] for p in mlps])
    W4 = jnp.stack([p[3][0][:, 0] for p in mlps])
    b4 = jnp.stack([p[3][1][0] for p in mlps]).reshape(1, NTYPE)

    A, B = _k1(h, WA, WB, n, 1000)
    X1 = _k2(A, B, src, dst, e)
    S = _k3(X1, ea.reshape(1, e), el.reshape(1, e), masks,
            w1ea, b1, W2, b2, W3, b3, W4, b4, e)
    EQ, MP = _k4(src, dst, ea, masks, S, coordT, n, e)
    coord_new, nep = _k5(h, coord, EQ, MP, params['attn'], n)
    return coord_new, nep
